# Initial kernel scaffold; baseline (speedup 1.0000x reference)
import jax
import jax.numpy as jnp
from jax.experimental import pallas as pl


def kernel(user_ids, item_ids, edge_index_u2i, edge_index_i2u, edge_label_index, user_emb, item_emb, Wsrc_u2i, Wdst_u2i, att_src_u2i, att_dst_u2i, b_u2i, Wsrc_i2u, Wdst_i2u, att_src_i2u, att_dst_i2u, b_i2u, Wl_user, bl_user, Wl_item, bl_item, Wd1, bd1, Wd2, bd2):
    raise NotImplementedError("write your pallas kernel here")


# speedup vs baseline: 15.4508x; 15.4508x over previous
import jax
import jax.numpy as jnp
from jax import lax
from jax.experimental import pallas as pl
from jax.experimental.pallas import tpu as pltpu
from jax.experimental.pallas import tpu_sc as plsc

NU = 50000
NI = 50000
D = 64
HALF = 32
E = 800000
EL = 160000
NPAD = 51200
SLC = 3200
CH = 256
NCH = E // CH
NCHL = EL // CH
R = 400
GRID = NU // R


def _mesh():
    return plsc.VectorSubcoreMesh(core_axis_name="c", subcore_axis_name="s",
                                  num_cores=2, num_subcores=16)


def _k1_body(ue_ref, ie_ref, wsu_ref, wdu_ref, asu_ref, adu_ref,
             wsi_ref, wdi_ref, asi_ref, adi_ref,
             hsu_ref, hsi_ref, a_ref, cs_ref, mx_ref):
    i = pl.program_id(0)
    ue = ue_ref[...]
    ie = ie_ref[...]
    hsu = jnp.dot(ue, wsu_ref[...], preferred_element_type=jnp.float32)
    hsi = jnp.dot(ie, wsi_ref[...], preferred_element_type=jnp.float32)
    hsu_ref[...] = jnp.stack([hsu[:, :HALF], hsu[:, HALF:]])
    hsi_ref[...] = jnp.stack([hsi[:, :HALF], hsi[:, HALF:]])
    asrc_u = jnp.dot(hsu, asu_ref[0])
    adst_u2i = jnp.dot(jnp.dot(ie, wdu_ref[...],
                               preferred_element_type=jnp.float32), adu_ref[0])
    asrc_i = jnp.dot(hsi, asi_ref[0])
    adst_i2u = jnp.dot(jnp.dot(ue, wdi_ref[...],
                               preferred_element_type=jnp.float32), adi_ref[0])
    a_ref[...] = jnp.stack([asrc_u, adst_u2i, asrc_i, adst_i2u], axis=1)
    m0 = jnp.max(asrc_u)
    m1 = jnp.max(adst_u2i)
    m2 = jnp.max(asrc_i)
    m3 = jnp.max(adst_i2u)

    @pl.when(i == 0)
    def _():
        mx_ref[0] = m0
        mx_ref[1] = m1
        mx_ref[2] = m2
        mx_ref[3] = m3

    @pl.when(i != 0)
    def _():
        mx_ref[0] = jnp.maximum(mx_ref[0], m0)
        mx_ref[1] = jnp.maximum(mx_ref[1], m1)
        mx_ref[2] = jnp.maximum(mx_ref[2], m2)
        mx_ref[3] = jnp.maximum(mx_ref[3], m3)

    c1 = mx_ref[0] + mx_ref[1]
    c1 = jnp.where(c1 > 0, c1, 0.2 * c1)
    c2 = mx_ref[2] + mx_ref[3]
    c2 = jnp.where(c2 > 0, c2, 0.2 * c2)
    cs_ref[...] = jnp.stack([jnp.full((128,), c1, jnp.float32),
                             jnp.full((128,), c2, jnp.float32)])


def _k1(user_emb, item_emb, wsu, wdu, asu, adu, wsi, wdi, asi, adi):
    wspec = pl.BlockSpec((D, D), lambda i: (0, 0))
    vspec = pl.BlockSpec((1, D), lambda i: (0, 0))
    return pl.pallas_call(
        _k1_body,
        grid=(GRID,),
        in_specs=[
            pl.BlockSpec((R, D), lambda i: (i, 0)),
            pl.BlockSpec((R, D), lambda i: (i, 0)),
            wspec, wspec, vspec, vspec, wspec, wspec, vspec, vspec,
        ],
        out_specs=[
            pl.BlockSpec((2, R, HALF), lambda i: (0, i, 0)),
            pl.BlockSpec((2, R, HALF), lambda i: (0, i, 0)),
            pl.BlockSpec((R, 4), lambda i: (i, 0)),
            pl.BlockSpec((2, 128), lambda i: (0, 0)),
        ],
        out_shape=[
            jax.ShapeDtypeStruct((2, NU, HALF), jnp.float32),
            jax.ShapeDtypeStruct((2, NI, HALF), jnp.float32),
            jax.ShapeDtypeStruct((NU, 4), jnp.float32),
            jax.ShapeDtypeStruct((2, 128), jnp.float32),
        ],
        scratch_shapes=[pltpu.SMEM((4,), jnp.float32)],
    )(user_emb, item_emb, wsu, wdu, asu.reshape(1, D), adu.reshape(1, D),
      wsi, wdi, asi.reshape(1, D), adi.reshape(1, D))


def _k2_body(s1, d1, s2, d2, as1, ad1, as2, ad2, cpack, zn_hbm,
             ex1, ex2, den10, den11, den20, den21,
             asrc_v, adst_v, cv, srcv, dstv, exbuf, den_sh):
    cid = lax.axis_index("c")
    sid = lax.axis_index("s")
    wid = cid * 16 + sid
    nw = jnp.where(wid < 21, 98, 97)
    pltpu.sync_copy(cpack, cv)
    for t in (0, 1):
        src_h = s1 if t == 0 else s2
        dst_h = d1 if t == 0 else d2
        ex_out = ex1 if t == 0 else ex2
        pltpu.sync_copy(as1 if t == 0 else as2, asrc_v)
        pltpu.sync_copy(ad1 if t == 0 else ad2, adst_v)
        pltpu.sync_copy(zn_hbm, den_sh.at[pl.ds(sid * SLC, SLC)])
        plsc.subcore_barrier()
        cval = cv[pl.ds(t * 16, 16)]

        def chunk_body(j, carry):
            base = (wid + j * 32) * CH
            for k in range(2):
                pltpu.sync_copy(src_h.at[pl.ds(base + k * 128, 128)],
                                srcv.at[k])
                pltpu.sync_copy(dst_h.at[pl.ds(base + k * 128, 128)],
                                dstv.at[k])
            for k in range(2):
                for g in range(8):
                    s16 = srcv[k, pl.ds(g * 16, 16)]
                    d16 = dstv[k, pl.ds(g * 16, 16)]
                    av = plsc.load_gather(asrc_v, [s16])
                    bv = plsc.load_gather(adst_v, [d16])
                    al = av + bv
                    al = jnp.where(al > 0, al, 0.2 * al)
                    exbuf[k, pl.ds(g * 16, 16)] = jnp.exp(al - cval)
            for k in range(2):
                pltpu.sync_copy(exbuf.at[k],
                                ex_out.at[pl.ds(base + k * 128, 128)])
                pltpu.sync_copy(exbuf.at[k], den_sh.at[dstv.at[k]], add=True)
            return carry

        lax.fori_loop(0, nw, chunk_body, 0)
        plsc.subcore_barrier()
        den_a = den10 if t == 0 else den20
        den_b = den11 if t == 0 else den21

        @pl.when(cid == 0)
        def _():
            pltpu.sync_copy(den_sh.at[pl.ds(sid * SLC, SLC)],
                            den_a.at[pl.ds(sid * SLC, SLC)])

        @pl.when(cid == 1)
        def _():
            pltpu.sync_copy(den_sh.at[pl.ds(sid * SLC, SLC)],
                            den_b.at[pl.ds(sid * SLC, SLC)])

        plsc.subcore_barrier()


def _k2(s1, d1, s2, d2, as1, ad1, as2, ad2, cpack, zerosn):
    f = pl.kernel(
        _k2_body,
        out_type=[
            jax.ShapeDtypeStruct((E,), jnp.float32),
            jax.ShapeDtypeStruct((E,), jnp.float32),
            jax.ShapeDtypeStruct((NPAD,), jnp.float32),
            jax.ShapeDtypeStruct((NPAD,), jnp.float32),
            jax.ShapeDtypeStruct((NPAD,), jnp.float32),
            jax.ShapeDtypeStruct((NPAD,), jnp.float32),
        ],
        mesh=_mesh(),
        compiler_params=pltpu.CompilerParams(needs_layout_passes=False, use_tc_tiling_on_sc=False),
        scratch_types=[
            pltpu.VMEM((NU,), jnp.float32),
            pltpu.VMEM((NU,), jnp.float32),
            pltpu.VMEM((32,), jnp.float32),
            pltpu.VMEM((2, 128), jnp.int32),
            pltpu.VMEM((2, 128), jnp.int32),
            pltpu.VMEM((2, 128), jnp.float32),
            pltpu.VMEM_SHARED((NPAD,), jnp.float32),
        ],
    )
    return f(s1, d1, s2, d2, as1, ad1, as2, ad2, cpack, zerosn)


def _k2b_body(d1, d2, ex1, ex2, den10, den11, den20, den21,
              w1, w2,
              dinv_v, ta, tb, dstv, exv, wbuf, dinv_sh):
    cid = lax.axis_index("c")
    sid = lax.axis_index("s")
    wid = cid * 16 + sid
    nw = jnp.where(wid < 21, 98, 97)
    for t in (0, 1):
        dst_h = d1 if t == 0 else d2
        ex_h = ex1 if t == 0 else ex2
        den_a = den10 if t == 0 else den20
        den_b = den11 if t == 0 else den21
        w_out = w1 if t == 0 else w2
        for i in range(4):
            off = sid * SLC + i * 800
            pltpu.sync_copy(den_a.at[pl.ds(off, 800)], ta)
            pltpu.sync_copy(den_b.at[pl.ds(off, 800)], tb)
            for g in range(50):
                v = ta[pl.ds(g * 16, 16)] + tb[pl.ds(g * 16, 16)]
                wbuf[pl.ds(g * 16, 16)] = 1.0 / (v + 1e-16)
            pltpu.sync_copy(wbuf.at[pl.ds(0, 800)],
                            dinv_sh.at[pl.ds(off, 800)])
        plsc.subcore_barrier()
        pltpu.sync_copy(dinv_sh, dinv_v)

        def chunk_body(j, carry):
            base = (wid + j * 32) * CH
            for k in range(2):
                pltpu.sync_copy(dst_h.at[pl.ds(base + k * 128, 128)],
                                dstv.at[k])
                pltpu.sync_copy(ex_h.at[pl.ds(base + k * 128, 128)],
                                exv.at[k])
            for k in range(2):
                for g in range(8):
                    d16 = dstv[k, pl.ds(g * 16, 16)]
                    winv = plsc.load_gather(dinv_v, [d16])
                    wbuf[pl.ds(k * 128 + g * 16, 16)] = (
                        exv[k, pl.ds(g * 16, 16)] * winv)
            pltpu.sync_copy(wbuf.at[pl.ds(0, CH)], w_out.at[pl.ds(base, CH)])
            return carry

        lax.fori_loop(0, nw, chunk_body, 0)
        plsc.subcore_barrier()


def _k2b(d1, d2, ex1, ex2, den10, den11, den20, den21):
    f = pl.kernel(
        _k2b_body,
        out_type=[
            jax.ShapeDtypeStruct((E,), jnp.float32),
            jax.ShapeDtypeStruct((E,), jnp.float32),
        ],
        mesh=_mesh(),
        compiler_params=pltpu.CompilerParams(needs_layout_passes=False, use_tc_tiling_on_sc=False),
        scratch_types=[
            pltpu.VMEM((NPAD,), jnp.float32),
            pltpu.VMEM((800,), jnp.float32),
            pltpu.VMEM((800,), jnp.float32),
            pltpu.VMEM((2, 128), jnp.int32),
            pltpu.VMEM((2, 128), jnp.float32),
            pltpu.VMEM((800,), jnp.float32),
            pltpu.VMEM_SHARED((NPAD,), jnp.float32),
        ],
    )
    return f(d1, d2, ex1, ex2, den10, den11, den20, den21)


def _k3_body(s1, d1, s2, d2, w1, w2, hu0, hu1, hi0, hi1,
             aggi_out, aggu_out,
             srcv, dstv, wv, rows, zbuf,
             acc_sh, sem):
    cid = lax.axis_index("c")
    sid = lax.axis_index("s")
    nw = jnp.where(sid < 5, 196, 195)

    def zb(r, carry):
        zbuf[r, pl.ds(0, 16)] = jnp.zeros((16,), jnp.float32)
        zbuf[r, pl.ds(16, 16)] = jnp.zeros((16,), jnp.float32)
        return carry

    lax.fori_loop(0, 100, zb, 0)

    for t in (0, 1):
        src_h = s1 if t == 0 else s2
        dst_h = d1 if t == 0 else d2
        w_h = w1 if t == 0 else w2
        tab0 = hu0 if t == 0 else hi0
        tab1 = hu1 if t == 0 else hi1
        agg = aggi_out if t == 0 else aggu_out

        for r in range(32):
            pltpu.sync_copy(zbuf,
                            acc_sh.at[pl.ds(sid * SLC + r * 100, 100), :])
        plsc.subcore_barrier()

        def chunk_body(j, carry):
            base = (sid + j * 16) * CH
            for k in range(2):
                pltpu.sync_copy(src_h.at[pl.ds(base + k * 128, 128)],
                                srcv.at[k])
                pltpu.sync_copy(dst_h.at[pl.ds(base + k * 128, 128)],
                                dstv.at[k])
                pltpu.sync_copy(w_h.at[pl.ds(base + k * 128, 128)],
                                wv.at[k])

            @pl.when(cid == 0)
            def _():
                g0 = pltpu.async_copy(tab0.at[srcv.at[0]], rows.at[0], sem)
                g1 = pltpu.async_copy(tab0.at[srcv.at[1]], rows.at[1], sem)
                g0.wait()
                g1.wait()

            @pl.when(cid == 1)
            def _():
                g0 = pltpu.async_copy(tab1.at[srcv.at[0]], rows.at[0], sem)
                g1 = pltpu.async_copy(tab1.at[srcv.at[1]], rows.at[1], sem)
                g0.wait()
                g1.wait()

            for k in range(2):
                def sb(g, carry2):
                    w16 = wv[k, pl.ds(g * 16, 16)]
                    for l in range(16):
                        wsb = jnp.full((16,), w16[l], jnp.float32)
                        e = g * 16 + l
                        rows[k, e, pl.ds(0, 16)] = (
                            rows[k, e, pl.ds(0, 16)] * wsb)
                        rows[k, e, pl.ds(16, 16)] = (
                            rows[k, e, pl.ds(16, 16)] * wsb)
                    return carry2

                lax.fori_loop(0, 8, sb, 0)
                pltpu.sync_copy(rows.at[k], acc_sh.at[dstv.at[k]], add=True)
            return carry

        lax.fori_loop(0, nw, chunk_body, 0)
        plsc.subcore_barrier()

        @pl.when(cid == 0)
        def _():
            pltpu.sync_copy(acc_sh.at[pl.ds(sid * SLC, SLC), :],
                            agg.at[pl.ds(sid * SLC, SLC), :])

        @pl.when(cid == 1)
        def _():
            pltpu.sync_copy(acc_sh.at[pl.ds(sid * SLC, SLC), :],
                            agg.at[pl.ds(NPAD + sid * SLC, SLC), :])

        plsc.subcore_barrier()


def _k3(s1, d1, s2, d2, w1, w2, hu0, hu1, hi0, hi1):
    f = pl.kernel(
        _k3_body,
        out_type=[
            jax.ShapeDtypeStruct((2 * NPAD, HALF), jnp.float32),
            jax.ShapeDtypeStruct((2 * NPAD, HALF), jnp.float32),
        ],
        mesh=_mesh(),
        compiler_params=pltpu.CompilerParams(needs_layout_passes=False, use_tc_tiling_on_sc=False),
        scratch_types=[
            pltpu.VMEM((2, 128), jnp.int32),
            pltpu.VMEM((2, 128), jnp.int32),
            pltpu.VMEM((2, 128), jnp.float32),
            pltpu.VMEM((2, 128, HALF), jnp.float32),
            pltpu.VMEM((100, HALF), jnp.float32),
            pltpu.VMEM_SHARED((NPAD, HALF), jnp.float32),
            pltpu.SemaphoreType.DMA,
        ],
    )
    return f(s1, d1, s2, d2, w1, w2, hu0, hu1, hi0, hi1)


def _k4_body(aggu0_ref, aggu1_ref, aggi0_ref, aggi1_ref,
             bu2i_ref, bi2u_ref, wlu_ref, blu_ref,
             wli_ref, bli_ref, wd1a_ref, wd1b_ref, bd1_ref,
             t_ref, s_ref):
    zu = jnp.concatenate([aggu0_ref[...], aggu1_ref[...]],
                         axis=-1) + bi2u_ref[0]
    tu = jax.nn.relu(jnp.dot(zu, wlu_ref[...],
                             preferred_element_type=jnp.float32) + blu_ref[0])
    t_ref[...] = jnp.dot(tu, wd1a_ref[...],
                         preferred_element_type=jnp.float32) + bd1_ref[0]
    zi = jnp.concatenate([aggi0_ref[...], aggi1_ref[...]],
                         axis=-1) + bu2i_ref[0]
    si = jax.nn.relu(jnp.dot(zi, wli_ref[...],
                             preferred_element_type=jnp.float32) + bli_ref[0])
    s_ref[...] = jnp.dot(si, wd1b_ref[...], preferred_element_type=jnp.float32)


def _k4(aggu, aggi, b_u2i, b_i2u, wlu, blu, wli, bli, wd1a, wd1b, bd1):
    wspec = pl.BlockSpec((D, D), lambda i: (0, 0))
    vspec = pl.BlockSpec((1, D), lambda i: (0, 0))
    h0spec = pl.BlockSpec((R, HALF), lambda i: (i, 0))
    h1spec = pl.BlockSpec((R, HALF), lambda i: (i + NPAD // R, 0))
    return pl.pallas_call(
        _k4_body,
        grid=(GRID,),
        in_specs=[h0spec, h1spec, h0spec, h1spec,
                  vspec, vspec, wspec, vspec, wspec, vspec,
                  wspec, wspec, vspec],
        out_specs=[pl.BlockSpec((R, D), lambda i: (i, 0)),
                   pl.BlockSpec((R, D), lambda i: (i, 0))],
        out_shape=[jax.ShapeDtypeStruct((NU, D), jnp.float32),
                   jax.ShapeDtypeStruct((NI, D), jnp.float32)],
    )(aggu, aggu, aggi, aggi, b_u2i.reshape(1, D), b_i2u.reshape(1, D),
      wlu, blu.reshape(1, D), wli, bli.reshape(1, D), wd1a, wd1b,
      bd1.reshape(1, D))


def _k5_body(rl, cl, t_hbm, s_hbm, wpack_hbm, out,
             rowv, colv, trows, srows, outbuf, wv, sem):
    cid = lax.axis_index("c")
    sid = lax.axis_index("s")
    wid = cid * 16 + sid
    nw = jnp.where(wid < 17, 20, 19)
    pltpu.sync_copy(wpack_hbm, wv)
    w0 = wv[pl.ds(0, 16)]
    w1 = wv[pl.ds(16, 16)]
    w2 = wv[pl.ds(32, 16)]
    w3 = wv[pl.ds(48, 16)]
    bd2s = wv[pl.ds(64, 16)][0]
    zero = jnp.zeros((16,), jnp.float32)

    def chunk_body(j, carry):
        base = (wid + j * 32) * CH
        for k in range(2):
            pltpu.sync_copy(rl.at[pl.ds(base + k * 128, 128)], rowv.at[k])
            pltpu.sync_copy(cl.at[pl.ds(base + k * 128, 128)], colv.at[k])
        g0 = pltpu.async_copy(t_hbm.at[rowv.at[0]], trows.at[0], sem)
        g1 = pltpu.async_copy(t_hbm.at[rowv.at[1]], trows.at[1], sem)
        g2 = pltpu.async_copy(s_hbm.at[colv.at[0]], srows.at[0], sem)
        g3 = pltpu.async_copy(s_hbm.at[colv.at[1]], srows.at[1], sem)
        g0.wait()
        g1.wait()
        g2.wait()
        g3.wait()
        for k in range(2):
            def eb(g, carry2):
                for l in range(4):
                    e = g * 4 + l
                    a0 = (trows[k, e, pl.ds(0, 16)] +
                          srows[k, e, pl.ds(0, 16)])
                    a1 = (trows[k, e, pl.ds(16, 16)] +
                          srows[k, e, pl.ds(16, 16)])
                    a2 = (trows[k, e, pl.ds(32, 16)] +
                          srows[k, e, pl.ds(32, 16)])
                    a3 = (trows[k, e, pl.ds(48, 16)] +
                          srows[k, e, pl.ds(48, 16)])
                    acc = (jnp.maximum(a0, zero) * w0 +
                           jnp.maximum(a1, zero) * w1 +
                           jnp.maximum(a2, zero) * w2 +
                           jnp.maximum(a3, zero) * w3)
                    o = jnp.sum(acc) + bd2s
                    plsc.store_scatter(
                        outbuf, [jnp.full((16,), k * 128 + e, jnp.int32)],
                        jnp.full((16,), o, jnp.float32))
                return carry2

            lax.fori_loop(0, 32, eb, 0)
        pltpu.sync_copy(outbuf, out.at[pl.ds(base, CH)])
        return carry

    lax.fori_loop(0, nw, chunk_body, 0)


def _k5(rl, cl, t_tab, s_tab, wpack):
    f = pl.kernel(
        _k5_body,
        out_type=jax.ShapeDtypeStruct((EL,), jnp.float32),
        mesh=_mesh(),
        compiler_params=pltpu.CompilerParams(needs_layout_passes=False, use_tc_tiling_on_sc=False),
        scratch_types=[
            pltpu.VMEM((2, 128), jnp.int32),
            pltpu.VMEM((2, 128), jnp.int32),
            pltpu.VMEM((2, 128, D), jnp.float32),
            pltpu.VMEM((2, 128, D), jnp.float32),
            pltpu.VMEM((CH,), jnp.float32),
            pltpu.VMEM((80,), jnp.float32),
            pltpu.SemaphoreType.DMA,
        ],
    )
    return f(rl, cl, t_tab, s_tab, wpack)


def kernel(user_ids, item_ids, edge_index_u2i, edge_index_i2u,
           edge_label_index, user_emb, item_emb,
           Wsrc_u2i, Wdst_u2i, att_src_u2i, att_dst_u2i, b_u2i,
           Wsrc_i2u, Wdst_i2u, att_src_i2u, att_dst_i2u, b_i2u,
           Wl_user, bl_user, Wl_item, bl_item,
           Wd1, bd1, Wd2, bd2):
    hsu_half, hsi_half, a_tab, cs = _k1(
        user_emb, item_emb, Wsrc_u2i, Wdst_u2i, att_src_u2i, att_dst_u2i,
        Wsrc_i2u, Wdst_i2u, att_src_i2u, att_dst_i2u)
    cpack = jnp.concatenate([jnp.broadcast_to(cs[0, :1], (16,)),
                             jnp.broadcast_to(cs[1, :1], (16,))])
    zerosn = jnp.zeros((SLC,), jnp.float32)
    s1, d1 = edge_index_u2i[0], edge_index_u2i[1]
    s2, d2 = edge_index_i2u[0], edge_index_i2u[1]
    ex1, ex2, den10, den11, den20, den21 = _k2(
        s1, d1, s2, d2, a_tab[:, 0], a_tab[:, 1], a_tab[:, 2], a_tab[:, 3],
        cpack, zerosn)
    w1, w2 = _k2b(d1, d2, ex1, ex2, den10, den11, den20, den21)
    aggi, aggu = _k3(s1, d1, s2, d2, w1, w2,
                     hsu_half[0], hsu_half[1], hsi_half[0], hsi_half[1])
    t_tab, s_tab = _k4(aggu, aggi, b_u2i, b_i2u, Wl_user, bl_user,
                       Wl_item, bl_item, Wd1[:D], Wd1[D:], bd1)
    wpack = jnp.concatenate([Wd2[:, 0],
                             jnp.broadcast_to(bd2, (16,))])
    res = _k5(edge_label_index[0], edge_label_index[1], t_tab, s_tab, wpack)
    return res.reshape(EL, 1)


# speedup vs baseline: 21.7459x; 1.4074x over previous
import jax
import jax.numpy as jnp
from jax import lax
from jax.experimental import pallas as pl
from jax.experimental.pallas import tpu as pltpu
from jax.experimental.pallas import tpu_sc as plsc

NU = 50000
NI = 50000
D = 64
HALF = 32
E = 800000
EL = 160000
NPAD = 51200
SLC = 3200
CH = 256
NCH = E // CH
NCHL = EL // CH
R = 400
GRID = NU // R


def _mesh():
    return plsc.VectorSubcoreMesh(core_axis_name="c", subcore_axis_name="s",
                                  num_cores=2, num_subcores=16)


def _k1_body(ue_ref, ie_ref, wsu_ref, wdu_ref, asu_ref, adu_ref,
             wsi_ref, wdi_ref, asi_ref, adi_ref,
             hsu_ref, hsi_ref, a_ref, cs_ref, mx_ref):
    i = pl.program_id(0)
    ue = ue_ref[...]
    ie = ie_ref[...]
    hsu = jnp.dot(ue, wsu_ref[...], preferred_element_type=jnp.float32)
    hsi = jnp.dot(ie, wsi_ref[...], preferred_element_type=jnp.float32)
    hsu_ref[...] = jnp.stack([hsu[:, :HALF], hsu[:, HALF:]])
    hsi_ref[...] = jnp.stack([hsi[:, :HALF], hsi[:, HALF:]])
    asrc_u = jnp.dot(hsu, asu_ref[0])
    adst_u2i = jnp.dot(jnp.dot(ie, wdu_ref[...],
                               preferred_element_type=jnp.float32), adu_ref[0])
    asrc_i = jnp.dot(hsi, asi_ref[0])
    adst_i2u = jnp.dot(jnp.dot(ue, wdi_ref[...],
                               preferred_element_type=jnp.float32), adi_ref[0])
    a_ref[...] = jnp.stack([asrc_u, adst_u2i, asrc_i, adst_i2u], axis=1)
    m0 = jnp.max(asrc_u)
    m1 = jnp.max(adst_u2i)
    m2 = jnp.max(asrc_i)
    m3 = jnp.max(adst_i2u)

    @pl.when(i == 0)
    def _():
        mx_ref[0] = m0
        mx_ref[1] = m1
        mx_ref[2] = m2
        mx_ref[3] = m3

    @pl.when(i != 0)
    def _():
        mx_ref[0] = jnp.maximum(mx_ref[0], m0)
        mx_ref[1] = jnp.maximum(mx_ref[1], m1)
        mx_ref[2] = jnp.maximum(mx_ref[2], m2)
        mx_ref[3] = jnp.maximum(mx_ref[3], m3)

    c1 = mx_ref[0] + mx_ref[1]
    c1 = jnp.where(c1 > 0, c1, 0.2 * c1)
    c2 = mx_ref[2] + mx_ref[3]
    c2 = jnp.where(c2 > 0, c2, 0.2 * c2)
    cs_ref[...] = jnp.stack([jnp.full((128,), c1, jnp.float32),
                             jnp.full((128,), c2, jnp.float32)])


def _k1(user_emb, item_emb, wsu, wdu, asu, adu, wsi, wdi, asi, adi):
    wspec = pl.BlockSpec((D, D), lambda i: (0, 0))
    vspec = pl.BlockSpec((1, D), lambda i: (0, 0))
    return pl.pallas_call(
        _k1_body,
        grid=(GRID,),
        in_specs=[
            pl.BlockSpec((R, D), lambda i: (i, 0)),
            pl.BlockSpec((R, D), lambda i: (i, 0)),
            wspec, wspec, vspec, vspec, wspec, wspec, vspec, vspec,
        ],
        out_specs=[
            pl.BlockSpec((2, R, HALF), lambda i: (0, i, 0)),
            pl.BlockSpec((2, R, HALF), lambda i: (0, i, 0)),
            pl.BlockSpec((R, 4), lambda i: (i, 0)),
            pl.BlockSpec((2, 128), lambda i: (0, 0)),
        ],
        out_shape=[
            jax.ShapeDtypeStruct((2, NU, HALF), jnp.float32),
            jax.ShapeDtypeStruct((2, NI, HALF), jnp.float32),
            jax.ShapeDtypeStruct((NU, 4), jnp.float32),
            jax.ShapeDtypeStruct((2, 128), jnp.float32),
        ],
        scratch_shapes=[pltpu.SMEM((4,), jnp.float32)],
    )(user_emb, item_emb, wsu, wdu, asu.reshape(1, D), adu.reshape(1, D),
      wsi, wdi, asi.reshape(1, D), adi.reshape(1, D))


def _k2_body(s1, d1, s2, d2, as1, ad1, as2, ad2, cpack, zn_hbm,
             ex1, ex2, den10, den11, den20, den21,
             asrc_v, adst_v, cv, srcv, dstv, exbuf, den_sh):
    cid = lax.axis_index("c")
    sid = lax.axis_index("s")
    wid = cid * 16 + sid
    nw = jnp.where(wid < 21, 98, 97)
    pltpu.sync_copy(cpack, cv)
    for t in (0, 1):
        src_h = s1 if t == 0 else s2
        dst_h = d1 if t == 0 else d2
        ex_out = ex1 if t == 0 else ex2
        pltpu.sync_copy(as1 if t == 0 else as2, asrc_v)
        pltpu.sync_copy(ad1 if t == 0 else ad2, adst_v)
        pltpu.sync_copy(zn_hbm, den_sh.at[pl.ds(sid * SLC, SLC)])
        plsc.subcore_barrier()
        cval = cv[pl.ds(t * 16, 16)]

        def chunk_body(j, carry):
            base = (wid + j * 32) * CH
            for k in range(2):
                pltpu.sync_copy(src_h.at[pl.ds(base + k * 128, 128)],
                                srcv.at[k])
                pltpu.sync_copy(dst_h.at[pl.ds(base + k * 128, 128)],
                                dstv.at[k])
            for k in range(2):
                for g in range(8):
                    s16 = srcv[k, pl.ds(g * 16, 16)]
                    d16 = dstv[k, pl.ds(g * 16, 16)]
                    av = plsc.load_gather(asrc_v, [s16])
                    bv = plsc.load_gather(adst_v, [d16])
                    al = av + bv
                    al = jnp.where(al > 0, al, 0.2 * al)
                    exbuf[k, pl.ds(g * 16, 16)] = jnp.exp(al - cval)
            for k in range(2):
                pltpu.sync_copy(exbuf.at[k],
                                ex_out.at[pl.ds(base + k * 128, 128)])
                pltpu.sync_copy(exbuf.at[k], den_sh.at[dstv.at[k]], add=True)
            return carry

        lax.fori_loop(0, nw, chunk_body, 0)
        plsc.subcore_barrier()
        den_a = den10 if t == 0 else den20
        den_b = den11 if t == 0 else den21

        @pl.when(cid == 0)
        def _():
            pltpu.sync_copy(den_sh.at[pl.ds(sid * SLC, SLC)],
                            den_a.at[pl.ds(sid * SLC, SLC)])

        @pl.when(cid == 1)
        def _():
            pltpu.sync_copy(den_sh.at[pl.ds(sid * SLC, SLC)],
                            den_b.at[pl.ds(sid * SLC, SLC)])

        plsc.subcore_barrier()


def _k2(s1, d1, s2, d2, as1, ad1, as2, ad2, cpack, zerosn):
    f = pl.kernel(
        _k2_body,
        out_type=[
            jax.ShapeDtypeStruct((E,), jnp.float32),
            jax.ShapeDtypeStruct((E,), jnp.float32),
            jax.ShapeDtypeStruct((NPAD,), jnp.float32),
            jax.ShapeDtypeStruct((NPAD,), jnp.float32),
            jax.ShapeDtypeStruct((NPAD,), jnp.float32),
            jax.ShapeDtypeStruct((NPAD,), jnp.float32),
        ],
        mesh=_mesh(),
        compiler_params=pltpu.CompilerParams(needs_layout_passes=False, use_tc_tiling_on_sc=False),
        scratch_types=[
            pltpu.VMEM((NU,), jnp.float32),
            pltpu.VMEM((NU,), jnp.float32),
            pltpu.VMEM((32,), jnp.float32),
            pltpu.VMEM((2, 128), jnp.int32),
            pltpu.VMEM((2, 128), jnp.int32),
            pltpu.VMEM((2, 128), jnp.float32),
            pltpu.VMEM_SHARED((NPAD,), jnp.float32),
        ],
    )
    return f(s1, d1, s2, d2, as1, ad1, as2, ad2, cpack, zerosn)


def _k2b_body(d1, d2, ex1, ex2, den10, den11, den20, den21,
              w1, w2,
              dinv_v, ta, tb, dstv, exv, wbuf, dinv_sh):
    cid = lax.axis_index("c")
    sid = lax.axis_index("s")
    wid = cid * 16 + sid
    nw = jnp.where(wid < 21, 98, 97)
    for t in (0, 1):
        dst_h = d1 if t == 0 else d2
        ex_h = ex1 if t == 0 else ex2
        den_a = den10 if t == 0 else den20
        den_b = den11 if t == 0 else den21
        w_out = w1 if t == 0 else w2
        for i in range(4):
            off = sid * SLC + i * 800
            pltpu.sync_copy(den_a.at[pl.ds(off, 800)], ta)
            pltpu.sync_copy(den_b.at[pl.ds(off, 800)], tb)
            for g in range(50):
                v = ta[pl.ds(g * 16, 16)] + tb[pl.ds(g * 16, 16)]
                wbuf[pl.ds(g * 16, 16)] = 1.0 / (v + 1e-16)
            pltpu.sync_copy(wbuf.at[pl.ds(0, 800)],
                            dinv_sh.at[pl.ds(off, 800)])
        plsc.subcore_barrier()
        pltpu.sync_copy(dinv_sh, dinv_v)

        def chunk_body(j, carry):
            base = (wid + j * 32) * CH
            for k in range(2):
                pltpu.sync_copy(dst_h.at[pl.ds(base + k * 128, 128)],
                                dstv.at[k])
                pltpu.sync_copy(ex_h.at[pl.ds(base + k * 128, 128)],
                                exv.at[k])
            for k in range(2):
                for g in range(8):
                    d16 = dstv[k, pl.ds(g * 16, 16)]
                    winv = plsc.load_gather(dinv_v, [d16])
                    wbuf[pl.ds(k * 128 + g * 16, 16)] = (
                        exv[k, pl.ds(g * 16, 16)] * winv)
            pltpu.sync_copy(wbuf.at[pl.ds(0, CH)], w_out.at[pl.ds(base, CH)])
            return carry

        lax.fori_loop(0, nw, chunk_body, 0)
        plsc.subcore_barrier()


def _k2b(d1, d2, ex1, ex2, den10, den11, den20, den21):
    f = pl.kernel(
        _k2b_body,
        out_type=[
            jax.ShapeDtypeStruct((E,), jnp.float32),
            jax.ShapeDtypeStruct((E,), jnp.float32),
        ],
        mesh=_mesh(),
        compiler_params=pltpu.CompilerParams(needs_layout_passes=False, use_tc_tiling_on_sc=False),
        scratch_types=[
            pltpu.VMEM((NPAD,), jnp.float32),
            pltpu.VMEM((800,), jnp.float32),
            pltpu.VMEM((800,), jnp.float32),
            pltpu.VMEM((2, 128), jnp.int32),
            pltpu.VMEM((2, 128), jnp.float32),
            pltpu.VMEM((800,), jnp.float32),
            pltpu.VMEM_SHARED((NPAD,), jnp.float32),
        ],
    )
    return f(d1, d2, ex1, ex2, den10, den11, den20, den21)


def _k3_body(s1, d1, s2, d2, w1, w2, hu0, hu1, hi0, hi1,
             aggi_out, aggu_out,
             srcv, dstv, wv, rows, zbuf,
             acc_sh, gs0, gs1, gs2, gs3, ssem):
    cid = lax.axis_index("c")
    sid = lax.axis_index("s")
    nw = jnp.where(sid < 10, 98, 97)
    gsems = (gs0, gs1, gs2, gs3)

    def zb(r, carry):
        zbuf[r, pl.ds(0, 16)] = jnp.zeros((16,), jnp.float32)
        zbuf[r, pl.ds(16, 16)] = jnp.zeros((16,), jnp.float32)
        return carry

    lax.fori_loop(0, 100, zb, 0)

    for t in (0, 1):
        src_h = s1 if t == 0 else s2
        dst_h = d1 if t == 0 else d2
        w_h = w1 if t == 0 else w2
        tab0 = hu0 if t == 0 else hi0
        tab1 = hu1 if t == 0 else hi1
        agg = aggi_out if t == 0 else aggu_out

        for r in range(32):
            pltpu.sync_copy(zbuf,
                            acc_sh.at[pl.ds(sid * SLC + r * 100, 100), :])
        plsc.subcore_barrier()

        def scale(k, w512):
            def sb(g, carry2):
                w16 = w512[pl.ds(k * 128 + g * 16, 16)]
                for l in range(16):
                    wsb = jnp.full((16,), w16[l], jnp.float32)
                    e = g * 16 + l
                    rows[k, e, pl.ds(0, 16)] = (
                        rows[k, e, pl.ds(0, 16)] * wsb)
                    rows[k, e, pl.ds(16, 16)] = (
                        rows[k, e, pl.ds(16, 16)] * wsb)
                return carry2

            lax.fori_loop(0, 8, sb, 0)

        def run_subchunks(tab, base, row0, nsub):
            pltpu.sync_copy(src_h.at[pl.ds(base, nsub * 128)],
                            srcv.at[pl.ds(0, nsub * 128)])
            pltpu.sync_copy(dst_h.at[pl.ds(row0, nsub), :],
                            dstv.at[pl.ds(0, nsub), :])
            pltpu.sync_copy(w_h.at[pl.ds(base, nsub * 128)],
                            wv.at[pl.ds(0, nsub * 128)])
            gds = [pltpu.async_copy(
                tab.at[srcv.at[pl.ds(k * 128, 128)]], rows.at[k], gsems[k])
                for k in range(nsub)]
            sds = []
            for k in range(nsub):
                gds[k].wait()
                scale(k, wv)
                sds.append(pltpu.async_copy(
                    rows.at[k], acc_sh.at[dstv.at[k]], ssem, add=True))
            for sd in sds:
                sd.wait()

        def pair_body(j, carry):
            p = sid + j * 16
            base = p * 512

            @pl.when(cid == 0)
            def _():
                run_subchunks(tab0, base, p * 4, 4)

            @pl.when(cid == 1)
            def _():
                run_subchunks(tab1, base, p * 4, 4)

            return carry

        lax.fori_loop(0, nw, pair_body, 0)

        @pl.when(sid == 15)
        def _():
            @pl.when(cid == 0)
            def _():
                run_subchunks(tab0, 3124 * 256, 3124 * 2, 2)

            @pl.when(cid == 1)
            def _():
                run_subchunks(tab1, 3124 * 256, 3124 * 2, 2)

        plsc.subcore_barrier()

        @pl.when(cid == 0)
        def _():
            pltpu.sync_copy(acc_sh.at[pl.ds(sid * SLC, SLC), :],
                            agg.at[pl.ds(sid * SLC, SLC), :])

        @pl.when(cid == 1)
        def _():
            pltpu.sync_copy(acc_sh.at[pl.ds(sid * SLC, SLC), :],
                            agg.at[pl.ds(NPAD + sid * SLC, SLC), :])

        plsc.subcore_barrier()


def _k3(s1, d1, s2, d2, w1, w2, hu0, hu1, hi0, hi1):
    f = pl.kernel(
        _k3_body,
        out_type=[
            jax.ShapeDtypeStruct((2 * NPAD, HALF), jnp.float32),
            jax.ShapeDtypeStruct((2 * NPAD, HALF), jnp.float32),
        ],
        mesh=_mesh(),
        compiler_params=pltpu.CompilerParams(needs_layout_passes=False, use_tc_tiling_on_sc=False),
        scratch_types=[
            pltpu.VMEM((512,), jnp.int32),
            pltpu.VMEM((4, 128), jnp.int32),
            pltpu.VMEM((512,), jnp.float32),
            pltpu.VMEM((4, 128, HALF), jnp.float32),
            pltpu.VMEM((100, HALF), jnp.float32),
            pltpu.VMEM_SHARED((NPAD, HALF), jnp.float32),
            pltpu.SemaphoreType.DMA,
            pltpu.SemaphoreType.DMA,
            pltpu.SemaphoreType.DMA,
            pltpu.SemaphoreType.DMA,
            pltpu.SemaphoreType.DMA,
        ],
    )
    return f(s1, d1, s2, d2, w1, w2, hu0, hu1, hi0, hi1)


def _k4_body(aggu0_ref, aggu1_ref, aggi0_ref, aggi1_ref,
             bu2i_ref, bi2u_ref, wlu_ref, blu_ref,
             wli_ref, bli_ref, wd1a_ref, wd1b_ref, bd1_ref,
             t_ref, s_ref):
    zu = jnp.concatenate([aggu0_ref[...], aggu1_ref[...]],
                         axis=-1) + bi2u_ref[0]
    tu = jax.nn.relu(jnp.dot(zu, wlu_ref[...],
                             preferred_element_type=jnp.float32) + blu_ref[0])
    t_ref[...] = jnp.dot(tu, wd1a_ref[...],
                         preferred_element_type=jnp.float32) + bd1_ref[0]
    zi = jnp.concatenate([aggi0_ref[...], aggi1_ref[...]],
                         axis=-1) + bu2i_ref[0]
    si = jax.nn.relu(jnp.dot(zi, wli_ref[...],
                             preferred_element_type=jnp.float32) + bli_ref[0])
    s_ref[...] = jnp.dot(si, wd1b_ref[...], preferred_element_type=jnp.float32)


def _k4(aggu, aggi, b_u2i, b_i2u, wlu, blu, wli, bli, wd1a, wd1b, bd1):
    wspec = pl.BlockSpec((D, D), lambda i: (0, 0))
    vspec = pl.BlockSpec((1, D), lambda i: (0, 0))
    h0spec = pl.BlockSpec((R, HALF), lambda i: (i, 0))
    h1spec = pl.BlockSpec((R, HALF), lambda i: (i + NPAD // R, 0))
    return pl.pallas_call(
        _k4_body,
        grid=(GRID,),
        in_specs=[h0spec, h1spec, h0spec, h1spec,
                  vspec, vspec, wspec, vspec, wspec, vspec,
                  wspec, wspec, vspec],
        out_specs=[pl.BlockSpec((R, D), lambda i: (i, 0)),
                   pl.BlockSpec((R, D), lambda i: (i, 0))],
        out_shape=[jax.ShapeDtypeStruct((NU, D), jnp.float32),
                   jax.ShapeDtypeStruct((NI, D), jnp.float32)],
    )(aggu, aggu, aggi, aggi, b_u2i.reshape(1, D), b_i2u.reshape(1, D),
      wlu, blu.reshape(1, D), wli, bli.reshape(1, D), wd1a, wd1b,
      bd1.reshape(1, D))


def _k5_body(rl, cl, t_hbm, s_hbm, wpack_hbm, out,
             rowv, colv, trows, srows, outbuf, wv, sem):
    cid = lax.axis_index("c")
    sid = lax.axis_index("s")
    wid = cid * 16 + sid
    nw = jnp.where(wid < 17, 20, 19)
    pltpu.sync_copy(wpack_hbm, wv)
    w0 = wv[pl.ds(0, 16)]
    w1 = wv[pl.ds(16, 16)]
    w2 = wv[pl.ds(32, 16)]
    w3 = wv[pl.ds(48, 16)]
    bd2s = wv[pl.ds(64, 16)][0]
    zero = jnp.zeros((16,), jnp.float32)

    def chunk_body(j, carry):
        base = (wid + j * 32) * CH
        for k in range(2):
            pltpu.sync_copy(rl.at[pl.ds(base + k * 128, 128)], rowv.at[k])
            pltpu.sync_copy(cl.at[pl.ds(base + k * 128, 128)], colv.at[k])
        g0 = pltpu.async_copy(t_hbm.at[rowv.at[0]], trows.at[0], sem)
        g1 = pltpu.async_copy(t_hbm.at[rowv.at[1]], trows.at[1], sem)
        g2 = pltpu.async_copy(s_hbm.at[colv.at[0]], srows.at[0], sem)
        g3 = pltpu.async_copy(s_hbm.at[colv.at[1]], srows.at[1], sem)
        g0.wait()
        g1.wait()
        g2.wait()
        g3.wait()
        for k in range(2):
            def eb(g, carry2):
                for l in range(4):
                    e = g * 4 + l
                    a0 = (trows[k, e, pl.ds(0, 16)] +
                          srows[k, e, pl.ds(0, 16)])
                    a1 = (trows[k, e, pl.ds(16, 16)] +
                          srows[k, e, pl.ds(16, 16)])
                    a2 = (trows[k, e, pl.ds(32, 16)] +
                          srows[k, e, pl.ds(32, 16)])
                    a3 = (trows[k, e, pl.ds(48, 16)] +
                          srows[k, e, pl.ds(48, 16)])
                    acc = (jnp.maximum(a0, zero) * w0 +
                           jnp.maximum(a1, zero) * w1 +
                           jnp.maximum(a2, zero) * w2 +
                           jnp.maximum(a3, zero) * w3)
                    o = jnp.sum(acc) + bd2s
                    plsc.store_scatter(
                        outbuf, [jnp.full((16,), k * 128 + e, jnp.int32)],
                        jnp.full((16,), o, jnp.float32))
                return carry2

            lax.fori_loop(0, 32, eb, 0)
        pltpu.sync_copy(outbuf, out.at[pl.ds(base, CH)])
        return carry

    lax.fori_loop(0, nw, chunk_body, 0)


def _k5(rl, cl, t_tab, s_tab, wpack):
    f = pl.kernel(
        _k5_body,
        out_type=jax.ShapeDtypeStruct((EL,), jnp.float32),
        mesh=_mesh(),
        compiler_params=pltpu.CompilerParams(needs_layout_passes=False, use_tc_tiling_on_sc=False),
        scratch_types=[
            pltpu.VMEM((2, 128), jnp.int32),
            pltpu.VMEM((2, 128), jnp.int32),
            pltpu.VMEM((2, 128, D), jnp.float32),
            pltpu.VMEM((2, 128, D), jnp.float32),
            pltpu.VMEM((CH,), jnp.float32),
            pltpu.VMEM((80,), jnp.float32),
            pltpu.SemaphoreType.DMA,
        ],
    )
    return f(rl, cl, t_tab, s_tab, wpack)


def kernel(user_ids, item_ids, edge_index_u2i, edge_index_i2u,
           edge_label_index, user_emb, item_emb,
           Wsrc_u2i, Wdst_u2i, att_src_u2i, att_dst_u2i, b_u2i,
           Wsrc_i2u, Wdst_i2u, att_src_i2u, att_dst_i2u, b_i2u,
           Wl_user, bl_user, Wl_item, bl_item,
           Wd1, bd1, Wd2, bd2):
    hsu_half, hsi_half, a_tab, cs = _k1(
        user_emb, item_emb, Wsrc_u2i, Wdst_u2i, att_src_u2i, att_dst_u2i,
        Wsrc_i2u, Wdst_i2u, att_src_i2u, att_dst_i2u)
    cpack = jnp.concatenate([jnp.broadcast_to(cs[0, :1], (16,)),
                             jnp.broadcast_to(cs[1, :1], (16,))])
    zerosn = jnp.zeros((SLC,), jnp.float32)
    s1, d1 = edge_index_u2i[0], edge_index_u2i[1]
    s2, d2 = edge_index_i2u[0], edge_index_i2u[1]
    ex1, ex2, den10, den11, den20, den21 = _k2(
        s1, d1, s2, d2, a_tab[:, 0], a_tab[:, 1], a_tab[:, 2], a_tab[:, 3],
        cpack, zerosn)
    w1, w2 = _k2b(d1, d2, ex1, ex2, den10, den11, den20, den21)
    aggi, aggu = _k3(s1, d1.reshape(E // 128, 128), s2,
                     d2.reshape(E // 128, 128), w1, w2,
                     hsu_half[0], hsu_half[1], hsi_half[0], hsi_half[1])
    t_tab, s_tab = _k4(aggu, aggi, b_u2i, b_i2u, Wl_user, bl_user,
                       Wl_item, bl_item, Wd1[:D], Wd1[D:], bd1)
    wpack = jnp.concatenate([Wd2[:, 0],
                             jnp.broadcast_to(bd2, (16,))])
    res = _k5(edge_label_index[0], edge_label_index[1], t_tab, s_tab, wpack)
    return res.reshape(EL, 1)


# speedup vs baseline: 28.3820x; 1.3052x over previous
import jax
import jax.numpy as jnp
from jax import lax
from jax.experimental import pallas as pl
from jax.experimental.pallas import tpu as pltpu
from jax.experimental.pallas import tpu_sc as plsc

NU = 50000
NI = 50000
D = 64
HALF = 32
E = 800000
EL = 160000
NPAD = 51200
SLC = 3200
CH = 256
NCH = E // CH
NCHL = EL // CH
R = 400
GRID = NU // R


def _mesh():
    return plsc.VectorSubcoreMesh(core_axis_name="c", subcore_axis_name="s",
                                  num_cores=2, num_subcores=16)


def _k1_body(ue_ref, ie_ref, wsu_ref, wdu_ref, asu_ref, adu_ref,
             wsi_ref, wdi_ref, asi_ref, adi_ref,
             hsu_ref, hsi_ref, a_ref, cs_ref, mx_ref):
    i = pl.program_id(0)
    ue = ue_ref[...]
    ie = ie_ref[...]
    hsu = jnp.dot(ue, wsu_ref[...], preferred_element_type=jnp.float32)
    hsi = jnp.dot(ie, wsi_ref[...], preferred_element_type=jnp.float32)
    hsu_ref[...] = jnp.stack([hsu[:, :HALF], hsu[:, HALF:]])
    hsi_ref[...] = jnp.stack([hsi[:, :HALF], hsi[:, HALF:]])
    asrc_u = jnp.dot(hsu, asu_ref[0])
    adst_u2i = jnp.dot(jnp.dot(ie, wdu_ref[...],
                               preferred_element_type=jnp.float32), adu_ref[0])
    asrc_i = jnp.dot(hsi, asi_ref[0])
    adst_i2u = jnp.dot(jnp.dot(ue, wdi_ref[...],
                               preferred_element_type=jnp.float32), adi_ref[0])
    a_ref[...] = jnp.stack([asrc_u, adst_u2i, asrc_i, adst_i2u], axis=1)
    m0 = jnp.max(asrc_u)
    m1 = jnp.max(adst_u2i)
    m2 = jnp.max(asrc_i)
    m3 = jnp.max(adst_i2u)

    @pl.when(i == 0)
    def _():
        mx_ref[0] = m0
        mx_ref[1] = m1
        mx_ref[2] = m2
        mx_ref[3] = m3

    @pl.when(i != 0)
    def _():
        mx_ref[0] = jnp.maximum(mx_ref[0], m0)
        mx_ref[1] = jnp.maximum(mx_ref[1], m1)
        mx_ref[2] = jnp.maximum(mx_ref[2], m2)
        mx_ref[3] = jnp.maximum(mx_ref[3], m3)

    c1 = mx_ref[0] + mx_ref[1]
    c1 = jnp.where(c1 > 0, c1, 0.2 * c1)
    c2 = mx_ref[2] + mx_ref[3]
    c2 = jnp.where(c2 > 0, c2, 0.2 * c2)
    cs_ref[...] = jnp.stack([jnp.full((128,), c1, jnp.float32),
                             jnp.full((128,), c2, jnp.float32)])


def _k1(user_emb, item_emb, wsu, wdu, asu, adu, wsi, wdi, asi, adi):
    wspec = pl.BlockSpec((D, D), lambda i: (0, 0))
    vspec = pl.BlockSpec((1, D), lambda i: (0, 0))
    return pl.pallas_call(
        _k1_body,
        grid=(GRID,),
        in_specs=[
            pl.BlockSpec((R, D), lambda i: (i, 0)),
            pl.BlockSpec((R, D), lambda i: (i, 0)),
            wspec, wspec, vspec, vspec, wspec, wspec, vspec, vspec,
        ],
        out_specs=[
            pl.BlockSpec((2, R, HALF), lambda i: (0, i, 0)),
            pl.BlockSpec((2, R, HALF), lambda i: (0, i, 0)),
            pl.BlockSpec((R, 4), lambda i: (i, 0)),
            pl.BlockSpec((2, 128), lambda i: (0, 0)),
        ],
        out_shape=[
            jax.ShapeDtypeStruct((2, NU, HALF), jnp.float32),
            jax.ShapeDtypeStruct((2, NI, HALF), jnp.float32),
            jax.ShapeDtypeStruct((NU, 4), jnp.float32),
            jax.ShapeDtypeStruct((2, 128), jnp.float32),
        ],
        scratch_shapes=[pltpu.SMEM((4,), jnp.float32)],
    )(user_emb, item_emb, wsu, wdu, asu.reshape(1, D), adu.reshape(1, D),
      wsi, wdi, asi.reshape(1, D), adi.reshape(1, D))


def _k2_body(s1, d1, s2, d2, as1, ad1, as2, ad2, cpack, zn_hbm,
             ex1, ex2, den10, den11, den20, den21,
             asrc_v, adst_v, cv, srcv, dstv, exbuf, den_sh, ssem):
    cid = lax.axis_index("c")
    sid = lax.axis_index("s")
    wid = cid * 16 + sid
    nw = jnp.where(wid < 26, 49, 48)
    pltpu.sync_copy(cpack, cv)
    for t in (0, 1):
        src_h = s1 if t == 0 else s2
        dst_h = d1 if t == 0 else d2
        ex_out = ex1 if t == 0 else ex2
        pltpu.sync_copy(as1 if t == 0 else as2, asrc_v)
        pltpu.sync_copy(ad1 if t == 0 else ad2, adst_v)
        pltpu.sync_copy(zn_hbm, den_sh.at[pl.ds(sid * SLC, SLC)])
        plsc.subcore_barrier()
        cval = cv[pl.ds(t * 16, 16)]

        def run_block(base, row0, nsub):
            pltpu.sync_copy(src_h.at[pl.ds(base, nsub * 128)],
                            srcv.at[pl.ds(0, nsub * 128)])
            pltpu.sync_copy(dst_h.at[pl.ds(row0, nsub), :],
                            dstv.at[pl.ds(0, nsub), :])
            for k in range(nsub):
                for g in range(8):
                    s16 = srcv[pl.ds(k * 128 + g * 16, 16)]
                    d16 = dstv[k, pl.ds(g * 16, 16)]
                    av = plsc.load_gather(asrc_v, [s16])
                    bv = plsc.load_gather(adst_v, [d16])
                    al = av + bv
                    al = jnp.where(al > 0, al, 0.2 * al)
                    exbuf[pl.ds(k * 128 + g * 16, 16)] = jnp.exp(al - cval)
            sds = [pltpu.async_copy(exbuf.at[pl.ds(k * 128, 128)],
                                    den_sh.at[dstv.at[k]], ssem, add=True)
                   for k in range(nsub)]
            pltpu.sync_copy(exbuf.at[pl.ds(0, nsub * 128)],
                            ex_out.at[pl.ds(base, nsub * 128)])
            for sd in sds:
                sd.wait()

        def chunk_body(j, carry):
            p = wid + j * 32
            run_block(p * 512, p * 4, 4)
            return carry

        lax.fori_loop(0, nw, chunk_body, 0)

        @pl.when(wid == 31)
        def _():
            run_block(3124 * 256, 3124 * 2, 2)

        plsc.subcore_barrier()
        den_a = den10 if t == 0 else den20
        den_b = den11 if t == 0 else den21

        @pl.when(cid == 0)
        def _():
            pltpu.sync_copy(den_sh.at[pl.ds(sid * SLC, SLC)],
                            den_a.at[pl.ds(sid * SLC, SLC)])

        @pl.when(cid == 1)
        def _():
            pltpu.sync_copy(den_sh.at[pl.ds(sid * SLC, SLC)],
                            den_b.at[pl.ds(sid * SLC, SLC)])

        plsc.subcore_barrier()


def _k2(s1, d1, s2, d2, as1, ad1, as2, ad2, cpack, zerosn):
    f = pl.kernel(
        _k2_body,
        out_type=[
            jax.ShapeDtypeStruct((E,), jnp.float32),
            jax.ShapeDtypeStruct((E,), jnp.float32),
            jax.ShapeDtypeStruct((NPAD,), jnp.float32),
            jax.ShapeDtypeStruct((NPAD,), jnp.float32),
            jax.ShapeDtypeStruct((NPAD,), jnp.float32),
            jax.ShapeDtypeStruct((NPAD,), jnp.float32),
        ],
        mesh=_mesh(),
        compiler_params=pltpu.CompilerParams(needs_layout_passes=False, use_tc_tiling_on_sc=False),
        scratch_types=[
            pltpu.VMEM((NU,), jnp.float32),
            pltpu.VMEM((NU,), jnp.float32),
            pltpu.VMEM((32,), jnp.float32),
            pltpu.VMEM((512,), jnp.int32),
            pltpu.VMEM((4, 128), jnp.int32),
            pltpu.VMEM((512,), jnp.float32),
            pltpu.VMEM_SHARED((NPAD,), jnp.float32),
            pltpu.SemaphoreType.DMA,
        ],
    )
    return f(s1, d1, s2, d2, as1, ad1, as2, ad2, cpack, zerosn)


def _k2b_body(d1, d2, ex1, ex2, den10, den11, den20, den21,
              w1, w2,
              dinv_v, ta, tb, dstv, exv, wbuf, dinv_sh):
    cid = lax.axis_index("c")
    sid = lax.axis_index("s")
    wid = cid * 16 + sid
    nw = jnp.where(wid < 26, 49, 48)
    for t in (0, 1):
        dst_h = d1 if t == 0 else d2
        ex_h = ex1 if t == 0 else ex2
        den_a = den10 if t == 0 else den20
        den_b = den11 if t == 0 else den21
        w_out = w1 if t == 0 else w2
        for i in range(4):
            off = sid * SLC + i * 800
            pltpu.sync_copy(den_a.at[pl.ds(off, 800)], ta)
            pltpu.sync_copy(den_b.at[pl.ds(off, 800)], tb)
            for g in range(50):
                v = ta[pl.ds(g * 16, 16)] + tb[pl.ds(g * 16, 16)]
                wbuf[pl.ds(g * 16, 16)] = 1.0 / (v + 1e-16)
            pltpu.sync_copy(wbuf.at[pl.ds(0, 800)],
                            dinv_sh.at[pl.ds(off, 800)])
        plsc.subcore_barrier()
        pltpu.sync_copy(dinv_sh, dinv_v)

        def run_block(base, n):
            pltpu.sync_copy(dst_h.at[pl.ds(base, n)], dstv.at[pl.ds(0, n)])
            pltpu.sync_copy(ex_h.at[pl.ds(base, n)], exv.at[pl.ds(0, n)])
            for g in range(n // 16):
                d16 = dstv[pl.ds(g * 16, 16)]
                winv = plsc.load_gather(dinv_v, [d16])
                wbuf[pl.ds(g * 16, 16)] = exv[pl.ds(g * 16, 16)] * winv
            pltpu.sync_copy(wbuf.at[pl.ds(0, n)], w_out.at[pl.ds(base, n)])

        def chunk_body(j, carry):
            run_block((wid + j * 32) * 512, 512)
            return carry

        lax.fori_loop(0, nw, chunk_body, 0)

        @pl.when(wid == 31)
        def _():
            run_block(3124 * 256, 256)

        plsc.subcore_barrier()


def _k2b(d1, d2, ex1, ex2, den10, den11, den20, den21):
    f = pl.kernel(
        _k2b_body,
        out_type=[
            jax.ShapeDtypeStruct((E,), jnp.float32),
            jax.ShapeDtypeStruct((E,), jnp.float32),
        ],
        mesh=_mesh(),
        compiler_params=pltpu.CompilerParams(needs_layout_passes=False, use_tc_tiling_on_sc=False),
        scratch_types=[
            pltpu.VMEM((NPAD,), jnp.float32),
            pltpu.VMEM((800,), jnp.float32),
            pltpu.VMEM((800,), jnp.float32),
            pltpu.VMEM((512,), jnp.int32),
            pltpu.VMEM((512,), jnp.float32),
            pltpu.VMEM((800,), jnp.float32),
            pltpu.VMEM_SHARED((NPAD,), jnp.float32),
        ],
    )
    return f(d1, d2, ex1, ex2, den10, den11, den20, den21)


def _k3_body(s1, d1, s2, d2, w1, w2, hu0, hu1, hi0, hi1,
             aggi_out, aggu_out,
             srcv, dstv, wv, rows, zbuf,
             acc_sh, gs0, gs1, gs2, gs3, ssem):
    cid = lax.axis_index("c")
    sid = lax.axis_index("s")
    nw = jnp.where(sid < 10, 98, 97)
    gsems = (gs0, gs1, gs2, gs3)

    def zb(r, carry):
        zbuf[r, pl.ds(0, 16)] = jnp.zeros((16,), jnp.float32)
        zbuf[r, pl.ds(16, 16)] = jnp.zeros((16,), jnp.float32)
        return carry

    lax.fori_loop(0, 100, zb, 0)

    for t in (0, 1):
        src_h = s1 if t == 0 else s2
        dst_h = d1 if t == 0 else d2
        w_h = w1 if t == 0 else w2
        tab0 = hu0 if t == 0 else hi0
        tab1 = hu1 if t == 0 else hi1
        agg = aggi_out if t == 0 else aggu_out

        for r in range(32):
            pltpu.sync_copy(zbuf,
                            acc_sh.at[pl.ds(sid * SLC + r * 100, 100), :])
        plsc.subcore_barrier()

        def scale(k, w512):
            def sb(g, carry2):
                w16 = w512[pl.ds(k * 128 + g * 16, 16)]
                for l in range(16):
                    wsb = jnp.full((16,), w16[l], jnp.float32)
                    e = g * 16 + l
                    rows[k, e, pl.ds(0, 16)] = (
                        rows[k, e, pl.ds(0, 16)] * wsb)
                    rows[k, e, pl.ds(16, 16)] = (
                        rows[k, e, pl.ds(16, 16)] * wsb)
                return carry2

            lax.fori_loop(0, 8, sb, 0)

        def run_subchunks(tab, base, row0, nsub):
            pltpu.sync_copy(src_h.at[pl.ds(base, nsub * 128)],
                            srcv.at[pl.ds(0, nsub * 128)])
            pltpu.sync_copy(dst_h.at[pl.ds(row0, nsub), :],
                            dstv.at[pl.ds(0, nsub), :])
            pltpu.sync_copy(w_h.at[pl.ds(base, nsub * 128)],
                            wv.at[pl.ds(0, nsub * 128)])
            gds = [pltpu.async_copy(
                tab.at[srcv.at[pl.ds(k * 128, 128)]], rows.at[k], gsems[k])
                for k in range(nsub)]
            sds = []
            for k in range(nsub):
                gds[k].wait()
                scale(k, wv)
                sds.append(pltpu.async_copy(
                    rows.at[k], acc_sh.at[dstv.at[k]], ssem, add=True))
            for sd in sds:
                sd.wait()

        def pair_body(j, carry):
            p = sid + j * 16
            base = p * 512

            @pl.when(cid == 0)
            def _():
                run_subchunks(tab0, base, p * 4, 4)

            @pl.when(cid == 1)
            def _():
                run_subchunks(tab1, base, p * 4, 4)

            return carry

        lax.fori_loop(0, nw, pair_body, 0)

        @pl.when(sid == 15)
        def _():
            @pl.when(cid == 0)
            def _():
                run_subchunks(tab0, 3124 * 256, 3124 * 2, 2)

            @pl.when(cid == 1)
            def _():
                run_subchunks(tab1, 3124 * 256, 3124 * 2, 2)

        plsc.subcore_barrier()

        @pl.when(cid == 0)
        def _():
            pltpu.sync_copy(acc_sh.at[pl.ds(sid * SLC, SLC), :],
                            agg.at[pl.ds(sid * SLC, SLC), :])

        @pl.when(cid == 1)
        def _():
            pltpu.sync_copy(acc_sh.at[pl.ds(sid * SLC, SLC), :],
                            agg.at[pl.ds(NPAD + sid * SLC, SLC), :])

        plsc.subcore_barrier()


def _k3(s1, d1, s2, d2, w1, w2, hu0, hu1, hi0, hi1):
    f = pl.kernel(
        _k3_body,
        out_type=[
            jax.ShapeDtypeStruct((2 * NPAD, HALF), jnp.float32),
            jax.ShapeDtypeStruct((2 * NPAD, HALF), jnp.float32),
        ],
        mesh=_mesh(),
        compiler_params=pltpu.CompilerParams(needs_layout_passes=False, use_tc_tiling_on_sc=False),
        scratch_types=[
            pltpu.VMEM((512,), jnp.int32),
            pltpu.VMEM((4, 128), jnp.int32),
            pltpu.VMEM((512,), jnp.float32),
            pltpu.VMEM((4, 128, HALF), jnp.float32),
            pltpu.VMEM((100, HALF), jnp.float32),
            pltpu.VMEM_SHARED((NPAD, HALF), jnp.float32),
            pltpu.SemaphoreType.DMA,
            pltpu.SemaphoreType.DMA,
            pltpu.SemaphoreType.DMA,
            pltpu.SemaphoreType.DMA,
            pltpu.SemaphoreType.DMA,
        ],
    )
    return f(s1, d1, s2, d2, w1, w2, hu0, hu1, hi0, hi1)


def _k4_body(aggu0_ref, aggu1_ref, aggi0_ref, aggi1_ref,
             bu2i_ref, bi2u_ref, wlu_ref, blu_ref,
             wli_ref, bli_ref, wd1a_ref, wd1b_ref, bd1_ref,
             t_ref, s_ref):
    zu = jnp.concatenate([aggu0_ref[...], aggu1_ref[...]],
                         axis=-1) + bi2u_ref[0]
    tu = jax.nn.relu(jnp.dot(zu, wlu_ref[...],
                             preferred_element_type=jnp.float32) + blu_ref[0])
    t_ref[...] = jnp.dot(tu, wd1a_ref[...],
                         preferred_element_type=jnp.float32) + bd1_ref[0]
    zi = jnp.concatenate([aggi0_ref[...], aggi1_ref[...]],
                         axis=-1) + bu2i_ref[0]
    si = jax.nn.relu(jnp.dot(zi, wli_ref[...],
                             preferred_element_type=jnp.float32) + bli_ref[0])
    s_ref[...] = jnp.dot(si, wd1b_ref[...], preferred_element_type=jnp.float32)


def _k4(aggu, aggi, b_u2i, b_i2u, wlu, blu, wli, bli, wd1a, wd1b, bd1):
    wspec = pl.BlockSpec((D, D), lambda i: (0, 0))
    vspec = pl.BlockSpec((1, D), lambda i: (0, 0))
    h0spec = pl.BlockSpec((R, HALF), lambda i: (i, 0))
    h1spec = pl.BlockSpec((R, HALF), lambda i: (i + NPAD // R, 0))
    return pl.pallas_call(
        _k4_body,
        grid=(GRID,),
        in_specs=[h0spec, h1spec, h0spec, h1spec,
                  vspec, vspec, wspec, vspec, wspec, vspec,
                  wspec, wspec, vspec],
        out_specs=[pl.BlockSpec((R, D), lambda i: (i, 0)),
                   pl.BlockSpec((R, D), lambda i: (i, 0))],
        out_shape=[jax.ShapeDtypeStruct((NU, D), jnp.float32),
                   jax.ShapeDtypeStruct((NI, D), jnp.float32)],
    )(aggu, aggu, aggi, aggi, b_u2i.reshape(1, D), b_i2u.reshape(1, D),
      wlu, blu.reshape(1, D), wli, bli.reshape(1, D), wd1a, wd1b,
      bd1.reshape(1, D))


def _k5_body(rl, cl, t_hbm, s_hbm, wpack_hbm, out,
             rowv, colv, trows, srows, outbuf, wv,
             gs0, gs1, gs2, gs3):
    cid = lax.axis_index("c")
    sid = lax.axis_index("s")
    wid = cid * 16 + sid
    nw = jnp.where(wid < 24, 10, 9)
    gsems = (gs0, gs1, gs2, gs3)
    pltpu.sync_copy(wpack_hbm, wv)
    w0 = wv[pl.ds(0, 16)]
    w1 = wv[pl.ds(16, 16)]
    w2 = wv[pl.ds(32, 16)]
    w3 = wv[pl.ds(48, 16)]
    bd2s = wv[pl.ds(64, 16)][0]
    zero = jnp.zeros((16,), jnp.float32)

    def run_block(base, nsub):
        pltpu.sync_copy(rl.at[pl.ds(base, nsub * 128)],
                        rowv.at[pl.ds(0, nsub * 128)])
        pltpu.sync_copy(cl.at[pl.ds(base, nsub * 128)],
                        colv.at[pl.ds(0, nsub * 128)])
        gds = []
        for k in range(nsub):
            gds.append(pltpu.async_copy(
                t_hbm.at[rowv.at[pl.ds(k * 128, 128)]], trows.at[k],
                gsems[k]))
            gds.append(pltpu.async_copy(
                s_hbm.at[colv.at[pl.ds(k * 128, 128)]], srows.at[k],
                gsems[k]))
        for k in range(nsub):
            gds[2 * k].wait()
            gds[2 * k + 1].wait()

            def eb(g, carry2):
                for l in range(4):
                    e = g * 4 + l
                    a0 = (trows[k, e, pl.ds(0, 16)] +
                          srows[k, e, pl.ds(0, 16)])
                    a1 = (trows[k, e, pl.ds(16, 16)] +
                          srows[k, e, pl.ds(16, 16)])
                    a2 = (trows[k, e, pl.ds(32, 16)] +
                          srows[k, e, pl.ds(32, 16)])
                    a3 = (trows[k, e, pl.ds(48, 16)] +
                          srows[k, e, pl.ds(48, 16)])
                    acc = (jnp.maximum(a0, zero) * w0 +
                           jnp.maximum(a1, zero) * w1 +
                           jnp.maximum(a2, zero) * w2 +
                           jnp.maximum(a3, zero) * w3)
                    o = jnp.sum(acc) + bd2s
                    plsc.store_scatter(
                        outbuf, [jnp.full((16,), k * 128 + e, jnp.int32)],
                        jnp.full((16,), o, jnp.float32))
                return carry2

            lax.fori_loop(0, 32, eb, 0)
        pltpu.sync_copy(outbuf.at[pl.ds(0, nsub * 128)],
                        out.at[pl.ds(base, nsub * 128)])

    def chunk_body(j, carry):
        run_block((wid + j * 32) * 512, 4)
        return carry

    lax.fori_loop(0, nw, chunk_body, 0)

    @pl.when(wid == 31)
    def _():
        run_block(624 * 256, 2)


def _k5(rl, cl, t_tab, s_tab, wpack):
    f = pl.kernel(
        _k5_body,
        out_type=jax.ShapeDtypeStruct((EL,), jnp.float32),
        mesh=_mesh(),
        compiler_params=pltpu.CompilerParams(needs_layout_passes=False, use_tc_tiling_on_sc=False),
        scratch_types=[
            pltpu.VMEM((512,), jnp.int32),
            pltpu.VMEM((512,), jnp.int32),
            pltpu.VMEM((4, 128, D), jnp.float32),
            pltpu.VMEM((4, 128, D), jnp.float32),
            pltpu.VMEM((512,), jnp.float32),
            pltpu.VMEM((80,), jnp.float32),
            pltpu.SemaphoreType.DMA,
            pltpu.SemaphoreType.DMA,
            pltpu.SemaphoreType.DMA,
            pltpu.SemaphoreType.DMA,
        ],
    )
    return f(rl, cl, t_tab, s_tab, wpack)


def kernel(user_ids, item_ids, edge_index_u2i, edge_index_i2u,
           edge_label_index, user_emb, item_emb,
           Wsrc_u2i, Wdst_u2i, att_src_u2i, att_dst_u2i, b_u2i,
           Wsrc_i2u, Wdst_i2u, att_src_i2u, att_dst_i2u, b_i2u,
           Wl_user, bl_user, Wl_item, bl_item,
           Wd1, bd1, Wd2, bd2):
    hsu_half, hsi_half, a_tab, cs = _k1(
        user_emb, item_emb, Wsrc_u2i, Wdst_u2i, att_src_u2i, att_dst_u2i,
        Wsrc_i2u, Wdst_i2u, att_src_i2u, att_dst_i2u)
    cpack = jnp.concatenate([jnp.broadcast_to(cs[0, :1], (16,)),
                             jnp.broadcast_to(cs[1, :1], (16,))])
    zerosn = jnp.zeros((SLC,), jnp.float32)
    s1, d1 = edge_index_u2i[0], edge_index_u2i[1]
    s2, d2 = edge_index_i2u[0], edge_index_i2u[1]
    d1r = d1.reshape(E // 128, 128)
    d2r = d2.reshape(E // 128, 128)
    ex1, ex2, den10, den11, den20, den21 = _k2(
        s1, d1r, s2, d2r, a_tab[:, 0], a_tab[:, 1], a_tab[:, 2], a_tab[:, 3],
        cpack, zerosn)
    w1, w2 = _k2b(d1, d2, ex1, ex2, den10, den11, den20, den21)
    aggi, aggu = _k3(s1, d1r, s2, d2r, w1, w2,
                     hsu_half[0], hsu_half[1], hsi_half[0], hsi_half[1])
    t_tab, s_tab = _k4(aggu, aggi, b_u2i, b_i2u, Wl_user, bl_user,
                       Wl_item, bl_item, Wd1[:D], Wd1[D:], bd1)
    wpack = jnp.concatenate([Wd2[:, 0],
                             jnp.broadcast_to(bd2, (16,))])
    res = _k5(edge_label_index[0], edge_label_index[1], t_tab, s_tab, wpack)
    return res.reshape(EL, 1)


# speedup vs baseline: 28.8696x; 1.0172x over previous
import jax
import jax.numpy as jnp
from jax import lax
from jax.experimental import pallas as pl
from jax.experimental.pallas import tpu as pltpu
from jax.experimental.pallas import tpu_sc as plsc

NU = 50000
NI = 50000
D = 64
HALF = 32
E = 800000
EL = 160000
NPAD = 51200
SLC = 3200
CH = 256
NCH = E // CH
NCHL = EL // CH
R = 400
GRID = NU // R


def _mesh():
    return plsc.VectorSubcoreMesh(core_axis_name="c", subcore_axis_name="s",
                                  num_cores=2, num_subcores=16)


def _k1_body(ue_ref, ie_ref, wsu_ref, wdu_ref, asu_ref, adu_ref,
             wsi_ref, wdi_ref, asi_ref, adi_ref,
             hsu_ref, hsi_ref, a_ref, cs_ref, mx_ref):
    i = pl.program_id(0)
    ue = ue_ref[...]
    ie = ie_ref[...]
    hsu = jnp.dot(ue, wsu_ref[...], preferred_element_type=jnp.float32)
    hsi = jnp.dot(ie, wsi_ref[...], preferred_element_type=jnp.float32)
    hsu_ref[...] = jnp.stack([hsu[:, :HALF], hsu[:, HALF:]])
    hsi_ref[...] = jnp.stack([hsi[:, :HALF], hsi[:, HALF:]])
    asrc_u = jnp.dot(hsu, asu_ref[0])
    adst_u2i = jnp.dot(jnp.dot(ie, wdu_ref[...],
                               preferred_element_type=jnp.float32), adu_ref[0])
    asrc_i = jnp.dot(hsi, asi_ref[0])
    adst_i2u = jnp.dot(jnp.dot(ue, wdi_ref[...],
                               preferred_element_type=jnp.float32), adi_ref[0])
    a_ref[...] = jnp.stack([asrc_u, adst_u2i, asrc_i, adst_i2u], axis=1)
    m0 = jnp.max(asrc_u)
    m1 = jnp.max(adst_u2i)
    m2 = jnp.max(asrc_i)
    m3 = jnp.max(adst_i2u)

    @pl.when(i == 0)
    def _():
        mx_ref[0] = m0
        mx_ref[1] = m1
        mx_ref[2] = m2
        mx_ref[3] = m3

    @pl.when(i != 0)
    def _():
        mx_ref[0] = jnp.maximum(mx_ref[0], m0)
        mx_ref[1] = jnp.maximum(mx_ref[1], m1)
        mx_ref[2] = jnp.maximum(mx_ref[2], m2)
        mx_ref[3] = jnp.maximum(mx_ref[3], m3)

    c1 = mx_ref[0] + mx_ref[1]
    c1 = jnp.where(c1 > 0, c1, 0.2 * c1)
    c2 = mx_ref[2] + mx_ref[3]
    c2 = jnp.where(c2 > 0, c2, 0.2 * c2)
    cs_ref[...] = jnp.stack([jnp.full((128,), c1, jnp.float32),
                             jnp.full((128,), c2, jnp.float32)])


def _k1(user_emb, item_emb, wsu, wdu, asu, adu, wsi, wdi, asi, adi):
    wspec = pl.BlockSpec((D, D), lambda i: (0, 0))
    vspec = pl.BlockSpec((1, D), lambda i: (0, 0))
    return pl.pallas_call(
        _k1_body,
        grid=(GRID,),
        in_specs=[
            pl.BlockSpec((R, D), lambda i: (i, 0)),
            pl.BlockSpec((R, D), lambda i: (i, 0)),
            wspec, wspec, vspec, vspec, wspec, wspec, vspec, vspec,
        ],
        out_specs=[
            pl.BlockSpec((2, R, HALF), lambda i: (0, i, 0)),
            pl.BlockSpec((2, R, HALF), lambda i: (0, i, 0)),
            pl.BlockSpec((R, 4), lambda i: (i, 0)),
            pl.BlockSpec((2, 128), lambda i: (0, 0)),
        ],
        out_shape=[
            jax.ShapeDtypeStruct((2, NU, HALF), jnp.float32),
            jax.ShapeDtypeStruct((2, NI, HALF), jnp.float32),
            jax.ShapeDtypeStruct((NU, 4), jnp.float32),
            jax.ShapeDtypeStruct((2, 128), jnp.float32),
        ],
        scratch_shapes=[pltpu.SMEM((4,), jnp.float32)],
    )(user_emb, item_emb, wsu, wdu, asu.reshape(1, D), adu.reshape(1, D),
      wsi, wdi, asi.reshape(1, D), adi.reshape(1, D))


def _k2_body(s1, d1r, s2, d2r, as1, ad1, as2, ad2, cpack, zn_hbm,
             ex1, ex2, w1, w2,
             asrc_v, adst_v, cv, srcv, dstv, exbuf, wv, ta, wb,
             den_sh, ssem):
    cid = lax.axis_index("c")
    sid = lax.axis_index("s")
    nw = jnp.where(sid < 10, 98, 97)
    pltpu.sync_copy(cpack, cv)
    for t in (0, 1):
        src_h = s1 if t == 0 else s2
        dst_h = d1r if t == 0 else d2r
        ex_out = ex1 if t == 0 else ex2
        w_out = w1 if t == 0 else w2
        asx = as1 if t == 0 else as2
        adx = ad1 if t == 0 else ad2

        @pl.when(cid == t)
        def _():
            pltpu.sync_copy(asx, asrc_v.at[pl.ds(0, NU)])
            pltpu.sync_copy(adx, adst_v)
            pltpu.sync_copy(zn_hbm, den_sh.at[pl.ds(sid * SLC, SLC)])
            plsc.subcore_barrier()
            cval = cv[pl.ds(t * 16, 16)]

            def run_block(base, row0, nsub):
                pltpu.sync_copy(src_h.at[pl.ds(base, nsub * 128)],
                                srcv.at[pl.ds(0, nsub * 128)])
                pltpu.sync_copy(dst_h.at[pl.ds(row0, nsub), :],
                                dstv.at[pl.ds(0, nsub), :])
                for k in range(nsub):
                    for g in range(8):
                        s16 = srcv[pl.ds(k * 128 + g * 16, 16)]
                        d16 = dstv[k, pl.ds(g * 16, 16)]
                        av = plsc.load_gather(asrc_v, [s16])
                        bv = plsc.load_gather(adst_v, [d16])
                        al = av + bv
                        al = jnp.where(al > 0, al, 0.2 * al)
                        exbuf[pl.ds(k * 128 + g * 16, 16)] = (
                            jnp.exp(al - cval))
                sds = [pltpu.async_copy(exbuf.at[pl.ds(k * 128, 128)],
                                        den_sh.at[dstv.at[k]], ssem,
                                        add=True)
                       for k in range(nsub)]
                pltpu.sync_copy(exbuf.at[pl.ds(0, nsub * 128)],
                                ex_out.at[pl.ds(base, nsub * 128)])
                for sd in sds:
                    sd.wait()

            def chunk_body(j, carry):
                p = sid + j * 16
                run_block(p * 512, p * 4, 4)
                return carry

            lax.fori_loop(0, nw, chunk_body, 0)

            @pl.when(sid == 15)
            def _():
                run_block(3124 * 256, 3124 * 2, 2)

            plsc.subcore_barrier()
            for i in range(4):
                off = sid * SLC + i * 800
                pltpu.sync_copy(den_sh.at[pl.ds(off, 800)], ta)
                for g in range(50):
                    v = ta[pl.ds(g * 16, 16)]
                    wb[pl.ds(g * 16, 16)] = 1.0 / (v + 1e-16)
                pltpu.sync_copy(wb, den_sh.at[pl.ds(off, 800)])
            plsc.subcore_barrier()
            pltpu.sync_copy(den_sh, asrc_v)

            def wrun_block(base, row0, nsub):
                pltpu.sync_copy(dst_h.at[pl.ds(row0, nsub), :],
                                dstv.at[pl.ds(0, nsub), :])
                pltpu.sync_copy(ex_out.at[pl.ds(base, nsub * 128)],
                                exbuf.at[pl.ds(0, nsub * 128)])
                for k in range(nsub):
                    for g in range(8):
                        d16 = dstv[k, pl.ds(g * 16, 16)]
                        winv = plsc.load_gather(asrc_v, [d16])
                        wv[pl.ds(k * 128 + g * 16, 16)] = (
                            exbuf[pl.ds(k * 128 + g * 16, 16)] * winv)
                pltpu.sync_copy(wv.at[pl.ds(0, nsub * 128)],
                                w_out.at[pl.ds(base, nsub * 128)])

            def wchunk_body(j, carry):
                p = sid + j * 16
                wrun_block(p * 512, p * 4, 4)
                return carry

            lax.fori_loop(0, nw, wchunk_body, 0)

            @pl.when(sid == 15)
            def _():
                wrun_block(3124 * 256, 3124 * 2, 2)


def _k2(s1, d1r, s2, d2r, as1, ad1, as2, ad2, cpack, zerosn):
    f = pl.kernel(
        _k2_body,
        out_type=[
            jax.ShapeDtypeStruct((E,), jnp.float32),
            jax.ShapeDtypeStruct((E,), jnp.float32),
            jax.ShapeDtypeStruct((E,), jnp.float32),
            jax.ShapeDtypeStruct((E,), jnp.float32),
        ],
        mesh=_mesh(),
        compiler_params=pltpu.CompilerParams(needs_layout_passes=False, use_tc_tiling_on_sc=False),
        scratch_types=[
            pltpu.VMEM((NPAD,), jnp.float32),
            pltpu.VMEM((NU,), jnp.float32),
            pltpu.VMEM((32,), jnp.float32),
            pltpu.VMEM((512,), jnp.int32),
            pltpu.VMEM((4, 128), jnp.int32),
            pltpu.VMEM((512,), jnp.float32),
            pltpu.VMEM((512,), jnp.float32),
            pltpu.VMEM((800,), jnp.float32),
            pltpu.VMEM((800,), jnp.float32),
            pltpu.VMEM_SHARED((NPAD,), jnp.float32),
            pltpu.SemaphoreType.DMA,
        ],
    )
    return f(s1, d1r, s2, d2r, as1, ad1, as2, ad2, cpack, zerosn)


def _k3_body(s1, d1, s2, d2, w1, w2, hu0, hu1, hi0, hi1,
             aggi_out, aggu_out,
             srcv, dstv, wv, rows, zbuf,
             acc_sh, gs0, gs1, gs2, gs3, ssem):
    cid = lax.axis_index("c")
    sid = lax.axis_index("s")
    nw = jnp.where(sid < 10, 98, 97)
    gsems = (gs0, gs1, gs2, gs3)

    def zb(r, carry):
        zbuf[r, pl.ds(0, 16)] = jnp.zeros((16,), jnp.float32)
        zbuf[r, pl.ds(16, 16)] = jnp.zeros((16,), jnp.float32)
        return carry

    lax.fori_loop(0, 100, zb, 0)

    for t in (0, 1):
        src_h = s1 if t == 0 else s2
        dst_h = d1 if t == 0 else d2
        w_h = w1 if t == 0 else w2
        tab0 = hu0 if t == 0 else hi0
        tab1 = hu1 if t == 0 else hi1
        agg = aggi_out if t == 0 else aggu_out

        for r in range(32):
            pltpu.sync_copy(zbuf,
                            acc_sh.at[pl.ds(sid * SLC + r * 100, 100), :])
        plsc.subcore_barrier()

        def scale(k, w512):
            def sb(g, carry2):
                w16 = w512[pl.ds(k * 128 + g * 16, 16)]
                for l in range(16):
                    wsb = jnp.full((16,), w16[l], jnp.float32)
                    e = g * 16 + l
                    rows[k, e, pl.ds(0, 16)] = (
                        rows[k, e, pl.ds(0, 16)] * wsb)
                    rows[k, e, pl.ds(16, 16)] = (
                        rows[k, e, pl.ds(16, 16)] * wsb)
                return carry2

            lax.fori_loop(0, 8, sb, 0)

        def run_subchunks(tab, base, row0, nsub):
            pltpu.sync_copy(src_h.at[pl.ds(base, nsub * 128)],
                            srcv.at[pl.ds(0, nsub * 128)])
            pltpu.sync_copy(dst_h.at[pl.ds(row0, nsub), :],
                            dstv.at[pl.ds(0, nsub), :])
            pltpu.sync_copy(w_h.at[pl.ds(base, nsub * 128)],
                            wv.at[pl.ds(0, nsub * 128)])
            gds = [pltpu.async_copy(
                tab.at[srcv.at[pl.ds(k * 128, 128)]], rows.at[k], gsems[k])
                for k in range(nsub)]
            sds = []
            for k in range(nsub):
                gds[k].wait()
                scale(k, wv)
                sds.append(pltpu.async_copy(
                    rows.at[k], acc_sh.at[dstv.at[k]], ssem, add=True))
            for sd in sds:
                sd.wait()

        def pair_body(j, carry):
            p = sid + j * 16
            base = p * 512

            @pl.when(cid == 0)
            def _():
                run_subchunks(tab0, base, p * 4, 4)

            @pl.when(cid == 1)
            def _():
                run_subchunks(tab1, base, p * 4, 4)

            return carry

        lax.fori_loop(0, nw, pair_body, 0)

        @pl.when(sid == 15)
        def _():
            @pl.when(cid == 0)
            def _():
                run_subchunks(tab0, 3124 * 256, 3124 * 2, 2)

            @pl.when(cid == 1)
            def _():
                run_subchunks(tab1, 3124 * 256, 3124 * 2, 2)

        plsc.subcore_barrier()

        @pl.when(cid == 0)
        def _():
            pltpu.sync_copy(acc_sh.at[pl.ds(sid * SLC, SLC), :],
                            agg.at[pl.ds(sid * SLC, SLC), :])

        @pl.when(cid == 1)
        def _():
            pltpu.sync_copy(acc_sh.at[pl.ds(sid * SLC, SLC), :],
                            agg.at[pl.ds(NPAD + sid * SLC, SLC), :])

        plsc.subcore_barrier()


def _k3(s1, d1, s2, d2, w1, w2, hu0, hu1, hi0, hi1):
    f = pl.kernel(
        _k3_body,
        out_type=[
            jax.ShapeDtypeStruct((2 * NPAD, HALF), jnp.float32),
            jax.ShapeDtypeStruct((2 * NPAD, HALF), jnp.float32),
        ],
        mesh=_mesh(),
        compiler_params=pltpu.CompilerParams(needs_layout_passes=False, use_tc_tiling_on_sc=False),
        scratch_types=[
            pltpu.VMEM((512,), jnp.int32),
            pltpu.VMEM((4, 128), jnp.int32),
            pltpu.VMEM((512,), jnp.float32),
            pltpu.VMEM((4, 128, HALF), jnp.float32),
            pltpu.VMEM((100, HALF), jnp.float32),
            pltpu.VMEM_SHARED((NPAD, HALF), jnp.float32),
            pltpu.SemaphoreType.DMA,
            pltpu.SemaphoreType.DMA,
            pltpu.SemaphoreType.DMA,
            pltpu.SemaphoreType.DMA,
            pltpu.SemaphoreType.DMA,
        ],
    )
    return f(s1, d1, s2, d2, w1, w2, hu0, hu1, hi0, hi1)


def _k4_body(aggu0_ref, aggu1_ref, aggi0_ref, aggi1_ref,
             bu2i_ref, bi2u_ref, wlu_ref, blu_ref,
             wli_ref, bli_ref, wd1a_ref, wd1b_ref, bd1_ref,
             t_ref, s_ref):
    zu = jnp.concatenate([aggu0_ref[...], aggu1_ref[...]],
                         axis=-1) + bi2u_ref[0]
    tu = jax.nn.relu(jnp.dot(zu, wlu_ref[...],
                             preferred_element_type=jnp.float32) + blu_ref[0])
    t_ref[...] = jnp.dot(tu, wd1a_ref[...],
                         preferred_element_type=jnp.float32) + bd1_ref[0]
    zi = jnp.concatenate([aggi0_ref[...], aggi1_ref[...]],
                         axis=-1) + bu2i_ref[0]
    si = jax.nn.relu(jnp.dot(zi, wli_ref[...],
                             preferred_element_type=jnp.float32) + bli_ref[0])
    s_ref[...] = jnp.dot(si, wd1b_ref[...], preferred_element_type=jnp.float32)


def _k4(aggu, aggi, b_u2i, b_i2u, wlu, blu, wli, bli, wd1a, wd1b, bd1):
    wspec = pl.BlockSpec((D, D), lambda i: (0, 0))
    vspec = pl.BlockSpec((1, D), lambda i: (0, 0))
    h0spec = pl.BlockSpec((R, HALF), lambda i: (i, 0))
    h1spec = pl.BlockSpec((R, HALF), lambda i: (i + NPAD // R, 0))
    return pl.pallas_call(
        _k4_body,
        grid=(GRID,),
        in_specs=[h0spec, h1spec, h0spec, h1spec,
                  vspec, vspec, wspec, vspec, wspec, vspec,
                  wspec, wspec, vspec],
        out_specs=[pl.BlockSpec((R, D), lambda i: (i, 0)),
                   pl.BlockSpec((R, D), lambda i: (i, 0))],
        out_shape=[jax.ShapeDtypeStruct((NU, D), jnp.float32),
                   jax.ShapeDtypeStruct((NI, D), jnp.float32)],
    )(aggu, aggu, aggi, aggi, b_u2i.reshape(1, D), b_i2u.reshape(1, D),
      wlu, blu.reshape(1, D), wli, bli.reshape(1, D), wd1a, wd1b,
      bd1.reshape(1, D))


def _k5_body(rl, cl, t_hbm, s_hbm, wpack_hbm, out,
             rowv, colv, trows, srows, outbuf, wv,
             gs0, gs1, gs2, gs3):
    cid = lax.axis_index("c")
    sid = lax.axis_index("s")
    wid = cid * 16 + sid
    nw = jnp.where(wid < 24, 10, 9)
    gsems = (gs0, gs1, gs2, gs3)
    pltpu.sync_copy(wpack_hbm, wv)
    w0 = wv[pl.ds(0, 16)]
    w1 = wv[pl.ds(16, 16)]
    w2 = wv[pl.ds(32, 16)]
    w3 = wv[pl.ds(48, 16)]
    bd2s = wv[pl.ds(64, 16)][0]
    zero = jnp.zeros((16,), jnp.float32)

    def run_block(base, nsub):
        pltpu.sync_copy(rl.at[pl.ds(base, nsub * 128)],
                        rowv.at[pl.ds(0, nsub * 128)])
        pltpu.sync_copy(cl.at[pl.ds(base, nsub * 128)],
                        colv.at[pl.ds(0, nsub * 128)])
        gds = []
        for k in range(nsub):
            gds.append(pltpu.async_copy(
                t_hbm.at[rowv.at[pl.ds(k * 128, 128)]], trows.at[k],
                gsems[k]))
            gds.append(pltpu.async_copy(
                s_hbm.at[colv.at[pl.ds(k * 128, 128)]], srows.at[k],
                gsems[k]))
        for k in range(nsub):
            gds[2 * k].wait()
            gds[2 * k + 1].wait()

            def eb(g, carry2):
                for l in range(4):
                    e = g * 4 + l
                    a0 = (trows[k, e, pl.ds(0, 16)] +
                          srows[k, e, pl.ds(0, 16)])
                    a1 = (trows[k, e, pl.ds(16, 16)] +
                          srows[k, e, pl.ds(16, 16)])
                    a2 = (trows[k, e, pl.ds(32, 16)] +
                          srows[k, e, pl.ds(32, 16)])
                    a3 = (trows[k, e, pl.ds(48, 16)] +
                          srows[k, e, pl.ds(48, 16)])
                    acc = (jnp.maximum(a0, zero) * w0 +
                           jnp.maximum(a1, zero) * w1 +
                           jnp.maximum(a2, zero) * w2 +
                           jnp.maximum(a3, zero) * w3)
                    o = jnp.sum(acc) + bd2s
                    plsc.store_scatter(
                        outbuf, [jnp.full((16,), k * 128 + e, jnp.int32)],
                        jnp.full((16,), o, jnp.float32))
                return carry2

            lax.fori_loop(0, 32, eb, 0)
        pltpu.sync_copy(outbuf.at[pl.ds(0, nsub * 128)],
                        out.at[pl.ds(base, nsub * 128)])

    def chunk_body(j, carry):
        run_block((wid + j * 32) * 512, 4)
        return carry

    lax.fori_loop(0, nw, chunk_body, 0)

    @pl.when(wid == 31)
    def _():
        run_block(624 * 256, 2)


def _k5(rl, cl, t_tab, s_tab, wpack):
    f = pl.kernel(
        _k5_body,
        out_type=jax.ShapeDtypeStruct((EL,), jnp.float32),
        mesh=_mesh(),
        compiler_params=pltpu.CompilerParams(needs_layout_passes=False, use_tc_tiling_on_sc=False),
        scratch_types=[
            pltpu.VMEM((512,), jnp.int32),
            pltpu.VMEM((512,), jnp.int32),
            pltpu.VMEM((4, 128, D), jnp.float32),
            pltpu.VMEM((4, 128, D), jnp.float32),
            pltpu.VMEM((512,), jnp.float32),
            pltpu.VMEM((80,), jnp.float32),
            pltpu.SemaphoreType.DMA,
            pltpu.SemaphoreType.DMA,
            pltpu.SemaphoreType.DMA,
            pltpu.SemaphoreType.DMA,
        ],
    )
    return f(rl, cl, t_tab, s_tab, wpack)


def kernel(user_ids, item_ids, edge_index_u2i, edge_index_i2u,
           edge_label_index, user_emb, item_emb,
           Wsrc_u2i, Wdst_u2i, att_src_u2i, att_dst_u2i, b_u2i,
           Wsrc_i2u, Wdst_i2u, att_src_i2u, att_dst_i2u, b_i2u,
           Wl_user, bl_user, Wl_item, bl_item,
           Wd1, bd1, Wd2, bd2):
    hsu_half, hsi_half, a_tab, cs = _k1(
        user_emb, item_emb, Wsrc_u2i, Wdst_u2i, att_src_u2i, att_dst_u2i,
        Wsrc_i2u, Wdst_i2u, att_src_i2u, att_dst_i2u)
    cpack = jnp.concatenate([jnp.broadcast_to(cs[0, :1], (16,)),
                             jnp.broadcast_to(cs[1, :1], (16,))])
    zerosn = jnp.zeros((SLC,), jnp.float32)
    s1, d1 = edge_index_u2i[0], edge_index_u2i[1]
    s2, d2 = edge_index_i2u[0], edge_index_i2u[1]
    d1r = d1.reshape(E // 128, 128)
    d2r = d2.reshape(E // 128, 128)
    ex1, ex2, w1, w2 = _k2(
        s1, d1r, s2, d2r, a_tab[:, 0], a_tab[:, 1], a_tab[:, 2], a_tab[:, 3],
        cpack, zerosn)
    aggi, aggu = _k3(s1, d1r, s2, d2r, w1, w2,
                     hsu_half[0], hsu_half[1], hsi_half[0], hsi_half[1])
    t_tab, s_tab = _k4(aggu, aggi, b_u2i, b_i2u, Wl_user, bl_user,
                       Wl_item, bl_item, Wd1[:D], Wd1[D:], bd1)
    wpack = jnp.concatenate([Wd2[:, 0],
                             jnp.broadcast_to(bd2, (16,))])
    res = _k5(edge_label_index[0], edge_label_index[1], t_tab, s_tab, wpack)
    return res.reshape(EL, 1)


# speedup vs baseline: 29.3893x; 1.0180x over previous
import jax
import jax.numpy as jnp
from jax import lax
from jax.experimental import pallas as pl
from jax.experimental.pallas import tpu as pltpu
from jax.experimental.pallas import tpu_sc as plsc

NU = 50000
NI = 50000
D = 64
HALF = 32
E = 800000
EL = 160000
NPAD = 51200
SLC = 3200
CH = 256
NCH = E // CH
NCHL = EL // CH
R = 400
GRID = NU // R


def _mesh():
    return plsc.VectorSubcoreMesh(core_axis_name="c", subcore_axis_name="s",
                                  num_cores=2, num_subcores=16)


def _k1_body(ue_ref, ie_ref, wsu_ref, wdu_ref, asu_ref, adu_ref,
             wsi_ref, wdi_ref, asi_ref, adi_ref,
             hu0_ref, hu1_ref, hi0_ref, hi1_ref, a_ref, cs_ref, mx_ref):
    i = pl.program_id(0)
    ue = ue_ref[...]
    ie = ie_ref[...]
    hsu = jnp.dot(ue, wsu_ref[...], preferred_element_type=jnp.float32)
    hsi = jnp.dot(ie, wsi_ref[...], preferred_element_type=jnp.float32)
    hu0_ref[...] = hsu[:, :HALF]
    hu1_ref[...] = hsu[:, HALF:]
    hi0_ref[...] = hsi[:, :HALF]
    hi1_ref[...] = hsi[:, HALF:]
    asrc_u = jnp.dot(hsu, asu_ref[0])
    adst_u2i = jnp.dot(jnp.dot(ie, wdu_ref[...],
                               preferred_element_type=jnp.float32), adu_ref[0])
    asrc_i = jnp.dot(hsi, asi_ref[0])
    adst_i2u = jnp.dot(jnp.dot(ue, wdi_ref[...],
                               preferred_element_type=jnp.float32), adi_ref[0])
    a_ref[...] = jnp.stack([asrc_u, adst_u2i, asrc_i, adst_i2u], axis=1)
    m0 = jnp.max(asrc_u)
    m1 = jnp.max(adst_u2i)
    m2 = jnp.max(asrc_i)
    m3 = jnp.max(adst_i2u)

    @pl.when(i == 0)
    def _():
        mx_ref[0] = m0
        mx_ref[1] = m1
        mx_ref[2] = m2
        mx_ref[3] = m3

    @pl.when(i != 0)
    def _():
        mx_ref[0] = jnp.maximum(mx_ref[0], m0)
        mx_ref[1] = jnp.maximum(mx_ref[1], m1)
        mx_ref[2] = jnp.maximum(mx_ref[2], m2)
        mx_ref[3] = jnp.maximum(mx_ref[3], m3)

    c1 = mx_ref[0] + mx_ref[1]
    c1 = jnp.where(c1 > 0, c1, 0.2 * c1)
    c2 = mx_ref[2] + mx_ref[3]
    c2 = jnp.where(c2 > 0, c2, 0.2 * c2)
    cs_ref[...] = jnp.stack([jnp.full((128,), c1, jnp.float32),
                             jnp.full((128,), c2, jnp.float32)])


def _k1(user_emb, item_emb, wsu, wdu, asu, adu, wsi, wdi, asi, adi):
    wspec = pl.BlockSpec((D, D), lambda i: (0, 0))
    vspec = pl.BlockSpec((1, D), lambda i: (0, 0))
    return pl.pallas_call(
        _k1_body,
        grid=(GRID,),
        in_specs=[
            pl.BlockSpec((R, D), lambda i: (i, 0)),
            pl.BlockSpec((R, D), lambda i: (i, 0)),
            wspec, wspec, vspec, vspec, wspec, wspec, vspec, vspec,
        ],
        out_specs=[
            pl.BlockSpec((R, HALF), lambda i: (i, 0)),
            pl.BlockSpec((R, HALF), lambda i: (i, 0)),
            pl.BlockSpec((R, HALF), lambda i: (i, 0)),
            pl.BlockSpec((R, HALF), lambda i: (i, 0)),
            pl.BlockSpec((R, 4), lambda i: (i, 0)),
            pl.BlockSpec((2, 128), lambda i: (0, 0)),
        ],
        out_shape=[
            jax.ShapeDtypeStruct((NU, HALF), jnp.float32),
            jax.ShapeDtypeStruct((NU, HALF), jnp.float32),
            jax.ShapeDtypeStruct((NI, HALF), jnp.float32),
            jax.ShapeDtypeStruct((NI, HALF), jnp.float32),
            jax.ShapeDtypeStruct((NU, 4), jnp.float32),
            jax.ShapeDtypeStruct((2, 128), jnp.float32),
        ],
        scratch_shapes=[pltpu.SMEM((4,), jnp.float32)],
    )(user_emb, item_emb, wsu, wdu, asu.reshape(1, D), adu.reshape(1, D),
      wsi, wdi, asi.reshape(1, D), adi.reshape(1, D))


def _k2_body(s1, d1r, s2, d2r, as1, ad1, as2, ad2, cpack, zn_hbm,
             ex1, ex2, w1, w2,
             asrc_v, adst_v, cv, srcv, dstv, exbuf, wv, ta, wb,
             den_sh, ssem):
    cid = lax.axis_index("c")
    sid = lax.axis_index("s")
    nw = jnp.where(sid < 10, 98, 97)
    pltpu.sync_copy(cpack, cv)
    for t in (0, 1):
        src_h = s1 if t == 0 else s2
        dst_h = d1r if t == 0 else d2r
        ex_out = ex1 if t == 0 else ex2
        w_out = w1 if t == 0 else w2
        asx = as1 if t == 0 else as2
        adx = ad1 if t == 0 else ad2

        @pl.when(cid == t)
        def _():
            pltpu.sync_copy(asx, asrc_v.at[pl.ds(0, NU)])
            pltpu.sync_copy(adx, adst_v)
            pltpu.sync_copy(zn_hbm, den_sh.at[pl.ds(sid * SLC, SLC)])
            plsc.subcore_barrier()
            cval = cv[pl.ds(t * 16, 16)]

            def run_block(base, row0, nsub):
                pltpu.sync_copy(src_h.at[pl.ds(base, nsub * 128)],
                                srcv.at[pl.ds(0, nsub * 128)])
                pltpu.sync_copy(dst_h.at[pl.ds(row0, nsub), :],
                                dstv.at[pl.ds(0, nsub), :])
                for k in range(nsub):
                    for g in range(8):
                        s16 = srcv[pl.ds(k * 128 + g * 16, 16)]
                        d16 = dstv[k, pl.ds(g * 16, 16)]
                        av = plsc.load_gather(asrc_v, [s16])
                        bv = plsc.load_gather(adst_v, [d16])
                        al = av + bv
                        al = jnp.where(al > 0, al, 0.2 * al)
                        exbuf[pl.ds(k * 128 + g * 16, 16)] = (
                            jnp.exp(al - cval))
                sds = [pltpu.async_copy(exbuf.at[pl.ds(k * 128, 128)],
                                        den_sh.at[dstv.at[k]], ssem,
                                        add=True)
                       for k in range(nsub)]
                pltpu.sync_copy(exbuf.at[pl.ds(0, nsub * 128)],
                                ex_out.at[pl.ds(base, nsub * 128)])
                for sd in sds:
                    sd.wait()

            def chunk_body(j, carry):
                p = sid + j * 16
                run_block(p * 512, p * 4, 4)
                return carry

            lax.fori_loop(0, nw, chunk_body, 0)

            @pl.when(sid == 15)
            def _():
                run_block(3124 * 256, 3124 * 2, 2)

            plsc.subcore_barrier()
            for i in range(4):
                off = sid * SLC + i * 800
                pltpu.sync_copy(den_sh.at[pl.ds(off, 800)], ta)
                for g in range(50):
                    v = ta[pl.ds(g * 16, 16)]
                    wb[pl.ds(g * 16, 16)] = 1.0 / (v + 1e-16)
                pltpu.sync_copy(wb, den_sh.at[pl.ds(off, 800)])
            plsc.subcore_barrier()
            pltpu.sync_copy(den_sh, asrc_v)

            def wrun_block(base, row0, nsub):
                pltpu.sync_copy(dst_h.at[pl.ds(row0, nsub), :],
                                dstv.at[pl.ds(0, nsub), :])
                pltpu.sync_copy(ex_out.at[pl.ds(base, nsub * 128)],
                                exbuf.at[pl.ds(0, nsub * 128)])
                for k in range(nsub):
                    for g in range(8):
                        d16 = dstv[k, pl.ds(g * 16, 16)]
                        winv = plsc.load_gather(asrc_v, [d16])
                        wv[pl.ds(k * 128 + g * 16, 16)] = (
                            exbuf[pl.ds(k * 128 + g * 16, 16)] * winv)
                pltpu.sync_copy(wv.at[pl.ds(0, nsub * 128)],
                                w_out.at[pl.ds(base, nsub * 128)])

            def wchunk_body(j, carry):
                p = sid + j * 16
                wrun_block(p * 512, p * 4, 4)
                return carry

            lax.fori_loop(0, nw, wchunk_body, 0)

            @pl.when(sid == 15)
            def _():
                wrun_block(3124 * 256, 3124 * 2, 2)


def _k2(s1, d1r, s2, d2r, as1, ad1, as2, ad2, cpack, zerosn):
    f = pl.kernel(
        _k2_body,
        out_type=[
            jax.ShapeDtypeStruct((E,), jnp.float32),
            jax.ShapeDtypeStruct((E,), jnp.float32),
            jax.ShapeDtypeStruct((E,), jnp.float32),
            jax.ShapeDtypeStruct((E,), jnp.float32),
        ],
        mesh=_mesh(),
        compiler_params=pltpu.CompilerParams(needs_layout_passes=False, use_tc_tiling_on_sc=False),
        scratch_types=[
            pltpu.VMEM((NPAD,), jnp.float32),
            pltpu.VMEM((NU,), jnp.float32),
            pltpu.VMEM((32,), jnp.float32),
            pltpu.VMEM((512,), jnp.int32),
            pltpu.VMEM((4, 128), jnp.int32),
            pltpu.VMEM((512,), jnp.float32),
            pltpu.VMEM((512,), jnp.float32),
            pltpu.VMEM((800,), jnp.float32),
            pltpu.VMEM((800,), jnp.float32),
            pltpu.VMEM_SHARED((NPAD,), jnp.float32),
            pltpu.SemaphoreType.DMA,
        ],
    )
    return f(s1, d1r, s2, d2r, as1, ad1, as2, ad2, cpack, zerosn)


def _k3_body(s1, d1, s2, d2, w1, w2, hu0, hu1, hi0, hi1,
             aggi_out, aggu_out,
             srcv, dstv, wv, rows, zbuf,
             acc_sh, gs0, gs1, gs2, gs3, ssem):
    cid = lax.axis_index("c")
    sid = lax.axis_index("s")
    nw = jnp.where(sid < 10, 98, 97)
    gsems = (gs0, gs1, gs2, gs3)

    def zb(r, carry):
        zbuf[r, pl.ds(0, 16)] = jnp.zeros((16,), jnp.float32)
        zbuf[r, pl.ds(16, 16)] = jnp.zeros((16,), jnp.float32)
        return carry

    lax.fori_loop(0, 100, zb, 0)

    for t in (0, 1):
        src_h = s1 if t == 0 else s2
        dst_h = d1 if t == 0 else d2
        w_h = w1 if t == 0 else w2
        tab0 = hu0 if t == 0 else hi0
        tab1 = hu1 if t == 0 else hi1
        agg = aggi_out if t == 0 else aggu_out

        for r in range(32):
            pltpu.sync_copy(zbuf,
                            acc_sh.at[pl.ds(sid * SLC + r * 100, 100), :])
        plsc.subcore_barrier()

        def scale(k, w512):
            def sb(g, carry2):
                w16 = w512[pl.ds(k * 128 + g * 16, 16)]
                for l in range(16):
                    wsb = jnp.full((16,), w16[l], jnp.float32)
                    e = g * 16 + l
                    rows[k, e, pl.ds(0, 16)] = (
                        rows[k, e, pl.ds(0, 16)] * wsb)
                    rows[k, e, pl.ds(16, 16)] = (
                        rows[k, e, pl.ds(16, 16)] * wsb)
                return carry2

            lax.fori_loop(0, 8, sb, 0)

        def run_subchunks(tab, base, row0, nsub):
            pltpu.sync_copy(src_h.at[pl.ds(base, nsub * 128)],
                            srcv.at[pl.ds(0, nsub * 128)])
            pltpu.sync_copy(dst_h.at[pl.ds(row0, nsub), :],
                            dstv.at[pl.ds(0, nsub), :])
            pltpu.sync_copy(w_h.at[pl.ds(base, nsub * 128)],
                            wv.at[pl.ds(0, nsub * 128)])
            gds = [pltpu.async_copy(
                tab.at[srcv.at[pl.ds(k * 128, 128)]], rows.at[k], gsems[k])
                for k in range(nsub)]
            sds = []
            for k in range(nsub):
                gds[k].wait()
                scale(k, wv)
                sds.append(pltpu.async_copy(
                    rows.at[k], acc_sh.at[dstv.at[k]], ssem, add=True))
            for sd in sds:
                sd.wait()

        def pair_body(j, carry):
            p = sid + j * 16
            base = p * 512

            @pl.when(cid == 0)
            def _():
                run_subchunks(tab0, base, p * 4, 4)

            @pl.when(cid == 1)
            def _():
                run_subchunks(tab1, base, p * 4, 4)

            return carry

        lax.fori_loop(0, nw, pair_body, 0)

        @pl.when(sid == 15)
        def _():
            @pl.when(cid == 0)
            def _():
                run_subchunks(tab0, 3124 * 256, 3124 * 2, 2)

            @pl.when(cid == 1)
            def _():
                run_subchunks(tab1, 3124 * 256, 3124 * 2, 2)

        plsc.subcore_barrier()

        @pl.when(cid == 0)
        def _():
            pltpu.sync_copy(acc_sh.at[pl.ds(sid * SLC, SLC), :],
                            agg.at[pl.ds(sid * SLC, SLC), :])

        @pl.when(cid == 1)
        def _():
            pltpu.sync_copy(acc_sh.at[pl.ds(sid * SLC, SLC), :],
                            agg.at[pl.ds(NPAD + sid * SLC, SLC), :])

        plsc.subcore_barrier()


def _k3(s1, d1, s2, d2, w1, w2, hu0, hu1, hi0, hi1):
    f = pl.kernel(
        _k3_body,
        out_type=[
            jax.ShapeDtypeStruct((2 * NPAD, HALF), jnp.float32),
            jax.ShapeDtypeStruct((2 * NPAD, HALF), jnp.float32),
        ],
        mesh=_mesh(),
        compiler_params=pltpu.CompilerParams(needs_layout_passes=False, use_tc_tiling_on_sc=False),
        scratch_types=[
            pltpu.VMEM((512,), jnp.int32),
            pltpu.VMEM((4, 128), jnp.int32),
            pltpu.VMEM((512,), jnp.float32),
            pltpu.VMEM((4, 128, HALF), jnp.float32),
            pltpu.VMEM((100, HALF), jnp.float32),
            pltpu.VMEM_SHARED((NPAD, HALF), jnp.float32),
            pltpu.SemaphoreType.DMA,
            pltpu.SemaphoreType.DMA,
            pltpu.SemaphoreType.DMA,
            pltpu.SemaphoreType.DMA,
            pltpu.SemaphoreType.DMA,
        ],
    )
    return f(s1, d1, s2, d2, w1, w2, hu0, hu1, hi0, hi1)


def _k4_body(aggu0_ref, aggu1_ref, aggi0_ref, aggi1_ref,
             bu2i_ref, bi2u_ref, wlu_ref, blu_ref,
             wli_ref, bli_ref, wd1a_ref, wd1b_ref, bd1_ref,
             t_ref, s_ref):
    zu = jnp.concatenate([aggu0_ref[...], aggu1_ref[...]],
                         axis=-1) + bi2u_ref[0]
    tu = jax.nn.relu(jnp.dot(zu, wlu_ref[...],
                             preferred_element_type=jnp.float32) + blu_ref[0])
    t_ref[...] = jnp.dot(tu, wd1a_ref[...],
                         preferred_element_type=jnp.float32) + bd1_ref[0]
    zi = jnp.concatenate([aggi0_ref[...], aggi1_ref[...]],
                         axis=-1) + bu2i_ref[0]
    si = jax.nn.relu(jnp.dot(zi, wli_ref[...],
                             preferred_element_type=jnp.float32) + bli_ref[0])
    s_ref[...] = jnp.dot(si, wd1b_ref[...], preferred_element_type=jnp.float32)


def _k4(aggu, aggi, b_u2i, b_i2u, wlu, blu, wli, bli, wd1a, wd1b, bd1):
    wspec = pl.BlockSpec((D, D), lambda i: (0, 0))
    vspec = pl.BlockSpec((1, D), lambda i: (0, 0))
    h0spec = pl.BlockSpec((R, HALF), lambda i: (i, 0))
    h1spec = pl.BlockSpec((R, HALF), lambda i: (i + NPAD // R, 0))
    return pl.pallas_call(
        _k4_body,
        grid=(GRID,),
        in_specs=[h0spec, h1spec, h0spec, h1spec,
                  vspec, vspec, wspec, vspec, wspec, vspec,
                  wspec, wspec, vspec],
        out_specs=[pl.BlockSpec((R, D), lambda i: (i, 0)),
                   pl.BlockSpec((R, D), lambda i: (i, 0))],
        out_shape=[jax.ShapeDtypeStruct((NU, D), jnp.float32),
                   jax.ShapeDtypeStruct((NI, D), jnp.float32)],
    )(aggu, aggu, aggi, aggi, b_u2i.reshape(1, D), b_i2u.reshape(1, D),
      wlu, blu.reshape(1, D), wli, bli.reshape(1, D), wd1a, wd1b,
      bd1.reshape(1, D))


def _k5_body(rl, cl, t_hbm, s_hbm, wpack_hbm, out,
             rowv, colv, trows, srows, outbuf, wv,
             gs0, gs1, gs2, gs3):
    cid = lax.axis_index("c")
    sid = lax.axis_index("s")
    wid = cid * 16 + sid
    nw = jnp.where(wid < 24, 10, 9)
    gsems = (gs0, gs1, gs2, gs3)
    pltpu.sync_copy(wpack_hbm, wv)
    w0 = wv[pl.ds(0, 16)]
    w1 = wv[pl.ds(16, 16)]
    w2 = wv[pl.ds(32, 16)]
    w3 = wv[pl.ds(48, 16)]
    bd2s = wv[pl.ds(64, 16)][0]
    zero = jnp.zeros((16,), jnp.float32)

    def run_block(base, nsub):
        pltpu.sync_copy(rl.at[pl.ds(base, nsub * 128)],
                        rowv.at[pl.ds(0, nsub * 128)])
        pltpu.sync_copy(cl.at[pl.ds(base, nsub * 128)],
                        colv.at[pl.ds(0, nsub * 128)])
        gds = []
        for k in range(nsub):
            gds.append(pltpu.async_copy(
                t_hbm.at[rowv.at[pl.ds(k * 128, 128)]], trows.at[k],
                gsems[k]))
            gds.append(pltpu.async_copy(
                s_hbm.at[colv.at[pl.ds(k * 128, 128)]], srows.at[k],
                gsems[k]))
        for k in range(nsub):
            gds[2 * k].wait()
            gds[2 * k + 1].wait()

            def eb(g, carry2):
                for l in range(4):
                    e = g * 4 + l
                    a0 = (trows[k, e, pl.ds(0, 16)] +
                          srows[k, e, pl.ds(0, 16)])
                    a1 = (trows[k, e, pl.ds(16, 16)] +
                          srows[k, e, pl.ds(16, 16)])
                    a2 = (trows[k, e, pl.ds(32, 16)] +
                          srows[k, e, pl.ds(32, 16)])
                    a3 = (trows[k, e, pl.ds(48, 16)] +
                          srows[k, e, pl.ds(48, 16)])
                    acc = (jnp.maximum(a0, zero) * w0 +
                           jnp.maximum(a1, zero) * w1 +
                           jnp.maximum(a2, zero) * w2 +
                           jnp.maximum(a3, zero) * w3)
                    o = jnp.sum(acc) + bd2s
                    plsc.store_scatter(
                        outbuf, [jnp.full((16,), k * 128 + e, jnp.int32)],
                        jnp.full((16,), o, jnp.float32))
                return carry2

            lax.fori_loop(0, 32, eb, 0)
        pltpu.sync_copy(outbuf.at[pl.ds(0, nsub * 128)],
                        out.at[pl.ds(base, nsub * 128)])

    def chunk_body(j, carry):
        run_block((wid + j * 32) * 512, 4)
        return carry

    lax.fori_loop(0, nw, chunk_body, 0)

    @pl.when(wid == 31)
    def _():
        run_block(624 * 256, 2)


def _k5(rl, cl, t_tab, s_tab, wpack):
    f = pl.kernel(
        _k5_body,
        out_type=jax.ShapeDtypeStruct((EL,), jnp.float32),
        mesh=_mesh(),
        compiler_params=pltpu.CompilerParams(needs_layout_passes=False, use_tc_tiling_on_sc=False),
        scratch_types=[
            pltpu.VMEM((512,), jnp.int32),
            pltpu.VMEM((512,), jnp.int32),
            pltpu.VMEM((4, 128, D), jnp.float32),
            pltpu.VMEM((4, 128, D), jnp.float32),
            pltpu.VMEM((512,), jnp.float32),
            pltpu.VMEM((80,), jnp.float32),
            pltpu.SemaphoreType.DMA,
            pltpu.SemaphoreType.DMA,
            pltpu.SemaphoreType.DMA,
            pltpu.SemaphoreType.DMA,
        ],
    )
    return f(rl, cl, t_tab, s_tab, wpack)


def kernel(user_ids, item_ids, edge_index_u2i, edge_index_i2u,
           edge_label_index, user_emb, item_emb,
           Wsrc_u2i, Wdst_u2i, att_src_u2i, att_dst_u2i, b_u2i,
           Wsrc_i2u, Wdst_i2u, att_src_i2u, att_dst_i2u, b_i2u,
           Wl_user, bl_user, Wl_item, bl_item,
           Wd1, bd1, Wd2, bd2):
    hu0, hu1, hi0, hi1, a_tab, cs = _k1(
        user_emb, item_emb, Wsrc_u2i, Wdst_u2i, att_src_u2i, att_dst_u2i,
        Wsrc_i2u, Wdst_i2u, att_src_i2u, att_dst_i2u)
    cpack = jnp.concatenate([jnp.broadcast_to(cs[0, :1], (16,)),
                             jnp.broadcast_to(cs[1, :1], (16,))])
    zerosn = jnp.zeros((SLC,), jnp.float32)
    s1, d1 = edge_index_u2i[0], edge_index_u2i[1]
    s2, d2 = edge_index_i2u[0], edge_index_i2u[1]
    d1r = d1.reshape(E // 128, 128)
    d2r = d2.reshape(E // 128, 128)
    ex1, ex2, w1, w2 = _k2(
        s1, d1r, s2, d2r, a_tab[:, 0], a_tab[:, 1], a_tab[:, 2], a_tab[:, 3],
        cpack, zerosn)
    aggi, aggu = _k3(s1, d1r, s2, d2r, w1, w2, hu0, hu1, hi0, hi1)
    t_tab, s_tab = _k4(aggu, aggi, b_u2i, b_i2u, Wl_user, bl_user,
                       Wl_item, bl_item, Wd1[:D], Wd1[D:], bd1)
    wpack = jnp.concatenate([Wd2[:, 0],
                             jnp.broadcast_to(bd2, (16,))])
    res = _k5(edge_label_index[0], edge_label_index[1], t_tab, s_tab, wpack)
    return res.reshape(EL, 1)
